# Initial kernel scaffold; baseline (speedup 1.0000x reference)
import jax
import jax.numpy as jnp
from jax.experimental import pallas as pl


def kernel(data, structure, W, b):
    raise NotImplementedError("write your pallas kernel here")


# speedup vs baseline: 13.3983x; 13.3983x over previous
import functools

import jax
import jax.numpy as jnp
from jax import lax
from jax.experimental import pallas as pl
from jax.experimental.pallas import tpu as pltpu
from jax.experimental.pallas import tpu_sc as plsc

N_NODES = 10000
D_FEAT = 256
N_EDGES = 160000
NPAD = 10240
ROWS128 = NPAD // 128
K_KEEP = N_NODES // 2
NTILES = 32
TILE_ROWS = 320
GROUPS16 = N_NODES // 16
EDGE_CHUNK = N_EDGES // NTILES
EDGE_STAGE = EDGE_CHUNK + 8
EPAD = N_EDGES + 16


def _mv_body(d_ref, w_ref, b_ref, o_ref):
    o_ref[...] = (
        jnp.dot(d_ref[...], w_ref[...], preferred_element_type=jnp.float32)
        + b_ref[0, 0]
    )


def _sel_body(s_ref, g_ref):
    s = s_ref[...]
    u = lax.bitcast_convert_type(s, jnp.uint32)
    key = jnp.where(u >= jnp.uint32(0x80000000), ~u, u | jnp.uint32(0x80000000))
    row = lax.broadcasted_iota(jnp.int32, (ROWS128, 128), 0)
    lane = lax.broadcasted_iota(jnp.int32, (ROWS128, 128), 1)
    flat = row * 128 + lane
    key = jnp.where(flat < N_NODES, key, jnp.uint32(0))

    def bs(i, ans):
        sh = jnp.uint32(31) - i.astype(jnp.uint32)
        cand = ans | (jnp.uint32(1) << sh)
        cnt = jnp.sum((key >= cand).astype(jnp.int32))
        return lax.select(cnt >= K_KEEP, cand, ans)

    thr = lax.fori_loop(0, 32, bs, jnp.uint32(0))
    cnt_gt = jnp.sum((key > thr).astype(jnp.int32))
    need = (K_KEEP - cnt_gt).astype(jnp.float32)
    eq = key == thr
    upper = (
        lax.broadcasted_iota(jnp.int32, (128, 128), 0)
        <= lax.broadcasted_iota(jnp.int32, (128, 128), 1)
    ).astype(jnp.float32)
    lstrict = (
        lax.broadcasted_iota(jnp.int32, (ROWS128, ROWS128), 0)
        > lax.broadcasted_iota(jnp.int32, (ROWS128, ROWS128), 1)
    ).astype(jnp.float32)
    ef = eq.astype(jnp.float32)
    incl_e = jnp.dot(ef, upper, preferred_element_type=jnp.float32)
    off_e = jnp.dot(lstrict, incl_e[:, 127:128], preferred_element_type=jnp.float32)
    excl_rank = incl_e - ef + off_e
    kept = (key > thr) | (eq & (excl_rank < need))
    kf = kept.astype(jnp.float32)
    incl_k = jnp.dot(kf, upper, preferred_element_type=jnp.float32)
    off_k = jnp.dot(lstrict, incl_k[:, 127:128], preferred_element_type=jnp.float32)
    new_id = (incl_k + off_k - 1.0).astype(jnp.int32)
    g_ref[...] = jnp.where(kept, new_id, -1)


def _scores_tc(data, W, b):
    return pl.pallas_call(
        _mv_body,
        grid=(8,),
        in_specs=[
            pl.BlockSpec((1280, D_FEAT), lambda i: (i, 0)),
            pl.BlockSpec((D_FEAT, 1), lambda i: (0, 0)),
            pl.BlockSpec((1, 1), lambda i: (0, 0)),
        ],
        out_specs=pl.BlockSpec((1280, 1), lambda i: (i, 0)),
        out_shape=jax.ShapeDtypeStruct((NPAD, 1), jnp.float32),
    )(data, W, b.reshape(1, 1))


def _select_tc(scores2d):
    return pl.pallas_call(
        _sel_body,
        out_shape=jax.ShapeDtypeStruct((ROWS128, 128), jnp.int32),
    )(scores2d)


def _sc_pool(data, scores, g, structp):
    mesh = plsc.VectorSubcoreMesh(core_axis_name="c", subcore_axis_name="s")

    @functools.partial(
        pl.kernel,
        mesh=mesh,
        compiler_params=pltpu.CompilerParams(needs_layout_passes=False),
        out_type=[
            jax.ShapeDtypeStruct((K_KEEP + 8, D_FEAT), jnp.float32),
            jax.ShapeDtypeStruct((EPAD,), jnp.int32),
            jax.ShapeDtypeStruct((EPAD,), jnp.int32),
        ],
        scratch_types=[
            pltpu.VMEM((TILE_ROWS, D_FEAT), jnp.float32),
            pltpu.VMEM((NPAD,), jnp.int32),
            pltpu.VMEM((TILE_ROWS,), jnp.float32),
            pltpu.VMEM((EDGE_STAGE,), jnp.int32),
            pltpu.VMEM((EDGE_STAGE,), jnp.int32),
            pltpu.VMEM((EDGE_STAGE,), jnp.int32),
            pltpu.VMEM((EDGE_STAGE,), jnp.int32),
            pltpu.SemaphoreType.DMA,
            pltpu.SemaphoreType.DMA,
            pltpu.SemaphoreType.DMA,
        ],
    )
    def k(data_hbm, scores_hbm, g_hbm, src_hbm, dst_hbm,
          pooled_hbm, ps0_hbm, ps1_hbm,
          dbuf, gt, sbuf, esrc, edst, eo0, eo1, sem_in, sem_e, sem_sc):
        wid = lax.axis_index("s") * 2 + lax.axis_index("c")
        g_start = (wid * GROUPS16) // NTILES
        r0 = g_start * 16
        e0 = wid * EDGE_CHUNK

        cp_data = pltpu.make_async_copy(
            data_hbm.at[pl.ds(r0, TILE_ROWS), :], dbuf, sem_in)
        cp_data.start()
        cp_src = pltpu.make_async_copy(
            src_hbm.at[pl.ds(e0, EDGE_STAGE)], esrc, sem_e)
        cp_src.start()
        cp_dst = pltpu.make_async_copy(
            dst_hbm.at[pl.ds(e0, EDGE_STAGE)], edst, sem_e)
        cp_dst.start()
        pltpu.sync_copy(g_hbm, gt)
        pltpu.sync_copy(scores_hbm.at[pl.ds(r0, TILE_ROWS)], sbuf)
        cp_data.wait()

        def grp_body(j, carry):
            svec = sbuf[pl.ds(j * 16, 16)]
            for r in range(16):
                vv = lax.broadcast(svec[r], (16,))
                row = j * 16 + r
                for c in range(D_FEAT // 16):
                    dbuf[row, pl.ds(c * 16, 16)] = (
                        dbuf[row, pl.ds(c * 16, 16)] * vv)
            return carry

        lax.fori_loop(0, TILE_ROWS // 16, grp_body, 0)

        scats = []
        for j in range(TILE_ROWS // 16):
            gv = gt[pl.ds(r0 + j * 16, 16)]
            idx = jnp.where(gv < 0, K_KEEP, gv)
            scats.append(pltpu.make_async_copy(
                dbuf.at[pl.ds(j * 16, 16), :], pooled_hbm.at[idx], sem_sc))
            scats[-1].start()

        cp_src.wait()
        cp_dst.wait()

        def ebody(i, carry):
            sv = esrc[pl.ds(i * 16, 16)]
            dv = edst[pl.ds(i * 16, 16)]
            sv = jnp.minimum(jnp.maximum(sv, 0), NPAD - 1)
            dv = jnp.minimum(jnp.maximum(dv, 0), NPAD - 1)
            a = plsc.load_gather(gt, [sv])
            bb = plsc.load_gather(gt, [dv])
            m = (a >= 0) & (bb >= 0)
            eo0[pl.ds(i * 16, 16)] = jnp.where(m, a, -1)
            eo1[pl.ds(i * 16, 16)] = jnp.where(m, bb, -1)
            return carry

        lax.fori_loop(0, EDGE_STAGE // 16, ebody, 0)
        pltpu.sync_copy(eo0, ps0_hbm.at[pl.ds(e0, EDGE_STAGE)])
        pltpu.sync_copy(eo1, ps1_hbm.at[pl.ds(e0, EDGE_STAGE)])
        for cp in scats:
            cp.wait()

    return k(data, scores, g, structp[0], structp[1])


def kernel(data, structure, W, b):
    scores2d = _scores_tc(data, W, b)
    g2d = _select_tc(scores2d.reshape(ROWS128, 128))
    structp = jnp.pad(structure, ((0, 0), (0, 16)))
    pooled, ps0, ps1 = _sc_pool(
        data, scores2d.reshape(NPAD), g2d.reshape(NPAD), structp)
    return pooled[:K_KEEP], jnp.stack([ps0[:N_EDGES], ps1[:N_EDGES]])


# speedup vs baseline: 40.2662x; 3.0053x over previous
import functools

import jax
import jax.numpy as jnp
from jax import lax
from jax.experimental import pallas as pl
from jax.experimental.pallas import tpu as pltpu
from jax.experimental.pallas import tpu_sc as plsc

N_NODES = 10000
D_FEAT = 256
N_EDGES = 160000
NPAD = 10240
ROWS128 = NPAD // 128
K_KEEP = N_NODES // 2
NTILES = 32
TILE_ROWS = 320
GROUPS16 = N_NODES // 16
EDGE_CHUNK = N_EDGES // NTILES
EDGE_STAGE = EDGE_CHUNK + 8
EPAD = N_EDGES + 16


def _mv_body(d_ref, w_ref, b_ref, o_ref):
    o_ref[...] = (
        jnp.dot(d_ref[...], w_ref[...], preferred_element_type=jnp.float32)
        + b_ref[0, 0]
    )


def _sel_body(s_ref, g_ref):
    s = s_ref[...]
    u = lax.bitcast_convert_type(s, jnp.uint32)
    key = jnp.where(u >= jnp.uint32(0x80000000), ~u, u | jnp.uint32(0x80000000))
    row = lax.broadcasted_iota(jnp.int32, (ROWS128, 128), 0)
    lane = lax.broadcasted_iota(jnp.int32, (ROWS128, 128), 1)
    flat = row * 128 + lane
    key = jnp.where(flat < N_NODES, key, jnp.uint32(0))

    def bs(i, ans):
        sh = jnp.uint32(31) - i.astype(jnp.uint32)
        cand = ans | (jnp.uint32(1) << sh)
        cnt = jnp.sum((key >= cand).astype(jnp.int32))
        return lax.select(cnt >= K_KEEP, cand, ans)

    thr = lax.fori_loop(0, 32, bs, jnp.uint32(0))
    cnt_gt = jnp.sum((key > thr).astype(jnp.int32))
    need = (K_KEEP - cnt_gt).astype(jnp.float32)
    eq = key == thr
    upper = (
        lax.broadcasted_iota(jnp.int32, (128, 128), 0)
        <= lax.broadcasted_iota(jnp.int32, (128, 128), 1)
    ).astype(jnp.float32)
    lstrict = (
        lax.broadcasted_iota(jnp.int32, (ROWS128, ROWS128), 0)
        > lax.broadcasted_iota(jnp.int32, (ROWS128, ROWS128), 1)
    ).astype(jnp.float32)
    ef = eq.astype(jnp.float32)
    incl_e = jnp.dot(ef, upper, preferred_element_type=jnp.float32)
    off_e = jnp.dot(lstrict, incl_e[:, 127:128], preferred_element_type=jnp.float32)
    excl_rank = incl_e - ef + off_e
    kept = (key > thr) | (eq & (excl_rank < need))
    kf = kept.astype(jnp.float32)
    incl_k = jnp.dot(kf, upper, preferred_element_type=jnp.float32)
    off_k = jnp.dot(lstrict, incl_k[:, 127:128], preferred_element_type=jnp.float32)
    new_id = (incl_k + off_k - 1.0).astype(jnp.int32)
    g_ref[...] = jnp.where(kept, new_id, -1)


def _scores_tc(data, W, b):
    return pl.pallas_call(
        _mv_body,
        grid=(8,),
        in_specs=[
            pl.BlockSpec((1280, D_FEAT), lambda i: (i, 0)),
            pl.BlockSpec((D_FEAT, 1), lambda i: (0, 0)),
            pl.BlockSpec((1, 1), lambda i: (0, 0)),
        ],
        out_specs=pl.BlockSpec((1280, 1), lambda i: (i, 0)),
        out_shape=jax.ShapeDtypeStruct((NPAD, 1), jnp.float32),
    )(data, W, b.reshape(1, 1))


def _select_tc(scores2d):
    return pl.pallas_call(
        _sel_body,
        out_shape=jax.ShapeDtypeStruct((ROWS128, 128), jnp.int32),
    )(scores2d)


def _sc_pool(data, scores, g, structp):
    mesh = plsc.VectorSubcoreMesh(core_axis_name="c", subcore_axis_name="s")

    @functools.partial(
        pl.kernel,
        mesh=mesh,
        compiler_params=pltpu.CompilerParams(needs_layout_passes=False),
        out_type=[
            jax.ShapeDtypeStruct((K_KEEP + NTILES, D_FEAT), jnp.float32),
            jax.ShapeDtypeStruct((EPAD,), jnp.int32),
            jax.ShapeDtypeStruct((EPAD,), jnp.int32),
        ],
        scratch_types=[
            pltpu.VMEM((TILE_ROWS, D_FEAT), jnp.float32),
            pltpu.VMEM((NPAD,), jnp.int32),
            pltpu.VMEM((TILE_ROWS,), jnp.float32),
            pltpu.VMEM((EDGE_STAGE,), jnp.int32),
            pltpu.VMEM((EDGE_STAGE,), jnp.int32),
            pltpu.VMEM((EDGE_STAGE,), jnp.int32),
            pltpu.VMEM((EDGE_STAGE,), jnp.int32),
            pltpu.SemaphoreType.DMA,
            pltpu.SemaphoreType.DMA,
            pltpu.SemaphoreType.DMA,
        ],
    )
    def k(data_hbm, scores_hbm, g_hbm, src_hbm, dst_hbm,
          pooled_hbm, ps0_hbm, ps1_hbm,
          dbuf, gt, sbuf, esrc, edst, eo0, eo1, sem_in, sem_e, sem_sc):
        wid = lax.axis_index("s") * 2 + lax.axis_index("c")
        g_start = (wid * GROUPS16) // NTILES
        r0 = g_start * 16
        e0 = wid * EDGE_CHUNK

        cp_data = pltpu.make_async_copy(
            data_hbm.at[pl.ds(r0, TILE_ROWS), :], dbuf, sem_in)
        cp_data.start()
        cp_src = pltpu.make_async_copy(
            src_hbm.at[pl.ds(e0, EDGE_STAGE)], esrc, sem_e)
        cp_src.start()
        cp_dst = pltpu.make_async_copy(
            dst_hbm.at[pl.ds(e0, EDGE_STAGE)], edst, sem_e)
        cp_dst.start()
        pltpu.sync_copy(g_hbm, gt)
        pltpu.sync_copy(scores_hbm.at[pl.ds(r0, TILE_ROWS)], sbuf)
        cp_data.wait()

        def grp_body(j, carry):
            svec = sbuf[pl.ds(j * 16, 16)]
            for r in range(16):
                vv = lax.broadcast(svec[r], (16,))
                row = j * 16 + r
                for c in range(D_FEAT // 16):
                    dbuf[row, pl.ds(c * 16, 16)] = (
                        dbuf[row, pl.ds(c * 16, 16)] * vv)
            return carry

        lax.fori_loop(0, TILE_ROWS // 16, grp_body, 0)

        dummy = K_KEEP + wid
        scats = []
        for j in range(TILE_ROWS // 16):
            gv = gt[pl.ds(r0 + j * 16, 16)]
            idx = jnp.where(gv < 0, dummy, gv)
            scats.append(pltpu.make_async_copy(
                dbuf.at[pl.ds(j * 16, 16), :], pooled_hbm.at[idx], sem_sc))
            scats[-1].start()

        cp_src.wait()
        cp_dst.wait()

        def ebody(i, carry):
            sv = esrc[pl.ds(i * 16, 16)]
            dv = edst[pl.ds(i * 16, 16)]
            sv = jnp.minimum(jnp.maximum(sv, 0), NPAD - 1)
            dv = jnp.minimum(jnp.maximum(dv, 0), NPAD - 1)
            a = plsc.load_gather(gt, [sv])
            bb = plsc.load_gather(gt, [dv])
            m = (a >= 0) & (bb >= 0)
            eo0[pl.ds(i * 16, 16)] = jnp.where(m, a, -1)
            eo1[pl.ds(i * 16, 16)] = jnp.where(m, bb, -1)
            return carry

        lax.fori_loop(0, EDGE_STAGE // 16, ebody, 0)
        pltpu.sync_copy(eo0, ps0_hbm.at[pl.ds(e0, EDGE_STAGE)])
        pltpu.sync_copy(eo1, ps1_hbm.at[pl.ds(e0, EDGE_STAGE)])
        for cp in scats:
            cp.wait()

    return k(data, scores, g, structp[0], structp[1])


def kernel(data, structure, W, b):
    scores2d = _scores_tc(data, W, b)
    g2d = _select_tc(scores2d.reshape(ROWS128, 128))
    structp = jnp.pad(structure, ((0, 0), (0, 16)))
    pooled, ps0, ps1 = _sc_pool(
        data, scores2d.reshape(NPAD), g2d.reshape(NPAD), structp)
    return pooled[:K_KEEP], jnp.stack([ps0[:N_EDGES], ps1[:N_EDGES]])


# speedup vs baseline: 43.2784x; 1.0748x over previous
import functools

import jax
import jax.numpy as jnp
from jax import lax
from jax.experimental import pallas as pl
from jax.experimental.pallas import tpu as pltpu
from jax.experimental.pallas import tpu_sc as plsc

N_NODES = 10000
D_FEAT = 256
N_EDGES = 160000
NPAD = 10240
ROWS128 = NPAD // 128
K_KEEP = N_NODES // 2
NTILES = 32
TILE_ROWS = 320
GROUPS16 = N_NODES // 16
EDGE_CHUNK = N_EDGES // NTILES
EDGE_STAGE = EDGE_CHUNK + 8
EPAD = N_EDGES + 16


def _mv_body(d_ref, w_ref, b_ref, o_ref):
    o_ref[...] = (
        jnp.dot(d_ref[...], w_ref[...], preferred_element_type=jnp.float32)
        + b_ref[0, 0]
    )


def _sel_body(s_ref, g_ref):
    s = s_ref[...]
    u = lax.bitcast_convert_type(s, jnp.uint32)
    key = jnp.where(u >= jnp.uint32(0x80000000), ~u, u | jnp.uint32(0x80000000))
    row = lax.broadcasted_iota(jnp.int32, (ROWS128, 128), 0)
    lane = lax.broadcasted_iota(jnp.int32, (ROWS128, 128), 1)
    flat = row * 128 + lane
    key = jnp.where(flat < N_NODES, key, jnp.uint32(0))

    def bs(i, ans):
        sh = jnp.uint32(31) - i.astype(jnp.uint32)
        cand = ans | (jnp.uint32(1) << sh)
        cnt = jnp.sum((key >= cand).astype(jnp.int32))
        return lax.select(cnt >= K_KEEP, cand, ans)

    thr = lax.fori_loop(0, 32, bs, jnp.uint32(0))
    cnt_gt = jnp.sum((key > thr).astype(jnp.int32))
    need = (K_KEEP - cnt_gt).astype(jnp.float32)
    eq = key == thr
    upper = (
        lax.broadcasted_iota(jnp.int32, (128, 128), 0)
        <= lax.broadcasted_iota(jnp.int32, (128, 128), 1)
    ).astype(jnp.float32)
    lstrict = (
        lax.broadcasted_iota(jnp.int32, (ROWS128, ROWS128), 0)
        > lax.broadcasted_iota(jnp.int32, (ROWS128, ROWS128), 1)
    ).astype(jnp.float32)
    ef = eq.astype(jnp.float32)
    incl_e = jnp.dot(ef, upper, preferred_element_type=jnp.float32)
    off_e = jnp.dot(lstrict, incl_e[:, 127:128], preferred_element_type=jnp.float32)
    excl_rank = incl_e - ef + off_e
    kept = (key > thr) | (eq & (excl_rank < need))
    kf = kept.astype(jnp.float32)
    incl_k = jnp.dot(kf, upper, preferred_element_type=jnp.float32)
    off_k = jnp.dot(lstrict, incl_k[:, 127:128], preferred_element_type=jnp.float32)
    new_id = (incl_k + off_k - 1.0).astype(jnp.int32)
    g_ref[...] = jnp.where(kept, new_id, -1)


def _scores_tc(data, W, b):
    return pl.pallas_call(
        _mv_body,
        grid=(8,),
        in_specs=[
            pl.BlockSpec((1280, D_FEAT), lambda i: (i, 0)),
            pl.BlockSpec((D_FEAT, 1), lambda i: (0, 0)),
            pl.BlockSpec((1, 1), lambda i: (0, 0)),
        ],
        out_specs=pl.BlockSpec((1280, 1), lambda i: (i, 0)),
        out_shape=jax.ShapeDtypeStruct((NPAD, 1), jnp.float32),
    )(data, W, b.reshape(1, 1))


def _select_tc(scores2d):
    return pl.pallas_call(
        _sel_body,
        out_shape=jax.ShapeDtypeStruct((ROWS128, 128), jnp.int32),
    )(scores2d)


def _sc_pool(data, scores, g, struct_src, struct_dst):
    mesh = plsc.VectorSubcoreMesh(core_axis_name="c", subcore_axis_name="s")

    @functools.partial(
        pl.kernel,
        mesh=mesh,
        compiler_params=pltpu.CompilerParams(needs_layout_passes=False),
        out_type=[
            jax.ShapeDtypeStruct((K_KEEP + NTILES, D_FEAT), jnp.float32),
            jax.ShapeDtypeStruct((2 * N_EDGES,), jnp.int32),
        ],
        scratch_types=[
            pltpu.VMEM((TILE_ROWS, D_FEAT), jnp.float32),
            pltpu.VMEM((NPAD,), jnp.int32),
            pltpu.VMEM((TILE_ROWS,), jnp.float32),
            pltpu.VMEM((EDGE_STAGE,), jnp.int32),
            pltpu.VMEM((EDGE_STAGE,), jnp.int32),
            pltpu.VMEM((EDGE_STAGE,), jnp.int32),
            pltpu.VMEM((EDGE_STAGE,), jnp.int32),
            pltpu.SemaphoreType.DMA,
            pltpu.SemaphoreType.DMA,
            pltpu.SemaphoreType.DMA,
        ],
    )
    def k(data_hbm, scores_hbm, g_hbm, src_hbm, dst_hbm,
          pooled_hbm, ps_hbm,
          dbuf, gt, sbuf, esrc, edst, eo0, eo1, sem_in, sem_e, sem_sc):
        wid = lax.axis_index("s") * 2 + lax.axis_index("c")
        g_start = (wid * GROUPS16) // NTILES
        r0 = g_start * 16
        e0 = wid * EDGE_CHUNK

        cp_data = pltpu.make_async_copy(
            data_hbm.at[pl.ds(r0, TILE_ROWS), :], dbuf, sem_in)
        cp_data.start()
        def _estage(n):
            return (
                pltpu.make_async_copy(
                    src_hbm.at[pl.ds(e0, n)], esrc.at[pl.ds(0, n)], sem_e),
                pltpu.make_async_copy(
                    dst_hbm.at[pl.ds(e0, n)], edst.at[pl.ds(0, n)], sem_e),
            )

        @pl.when(wid < NTILES - 1)
        def _():
            for cp in _estage(EDGE_STAGE):
                cp.start()

        @pl.when(wid == NTILES - 1)
        def _():
            for cp in _estage(EDGE_CHUNK):
                cp.start()
        pltpu.sync_copy(g_hbm, gt)
        pltpu.sync_copy(scores_hbm.at[pl.ds(r0, TILE_ROWS)], sbuf)
        cp_data.wait()

        def grp_body(j, carry):
            svec = sbuf[pl.ds(j * 16, 16)]
            for r in range(16):
                vv = lax.broadcast(svec[r], (16,))
                row = j * 16 + r
                for c in range(D_FEAT // 16):
                    dbuf[row, pl.ds(c * 16, 16)] = (
                        dbuf[row, pl.ds(c * 16, 16)] * vv)
            return carry

        lax.fori_loop(0, TILE_ROWS // 16, grp_body, 0)

        dummy = K_KEEP + wid
        scats = []
        for j in range(TILE_ROWS // 16):
            gv = gt[pl.ds(r0 + j * 16, 16)]
            idx = jnp.where(gv < 0, dummy, gv)
            scats.append(pltpu.make_async_copy(
                dbuf.at[pl.ds(j * 16, 16), :], pooled_hbm.at[idx], sem_sc))
            scats[-1].start()

        @pl.when(wid < NTILES - 1)
        def _():
            for cp in _estage(EDGE_STAGE):
                cp.wait()

        @pl.when(wid == NTILES - 1)
        def _():
            for cp in _estage(EDGE_CHUNK):
                cp.wait()

        def ebody(i, carry):
            sv = esrc[pl.ds(i * 16, 16)]
            dv = edst[pl.ds(i * 16, 16)]
            sv = jnp.minimum(jnp.maximum(sv, 0), NPAD - 1)
            dv = jnp.minimum(jnp.maximum(dv, 0), NPAD - 1)
            a = plsc.load_gather(gt, [sv])
            bb = plsc.load_gather(gt, [dv])
            m = (a >= 0) & (bb >= 0)
            eo0[pl.ds(i * 16, 16)] = jnp.where(m, a, -1)
            eo1[pl.ds(i * 16, 16)] = jnp.where(m, bb, -1)
            return carry

        lax.fori_loop(0, EDGE_STAGE // 16, ebody, 0)

        @pl.when(wid < NTILES - 1)
        def _():
            pltpu.sync_copy(eo0.at[pl.ds(0, EDGE_STAGE)],
                            ps_hbm.at[pl.ds(e0, EDGE_STAGE)])
            pltpu.sync_copy(eo1.at[pl.ds(0, EDGE_STAGE)],
                            ps_hbm.at[pl.ds(N_EDGES + e0, EDGE_STAGE)])

        @pl.when(wid == NTILES - 1)
        def _():
            pltpu.sync_copy(eo0.at[pl.ds(0, EDGE_CHUNK)],
                            ps_hbm.at[pl.ds(e0, EDGE_CHUNK)])
            pltpu.sync_copy(eo1.at[pl.ds(0, EDGE_CHUNK)],
                            ps_hbm.at[pl.ds(N_EDGES + e0, EDGE_CHUNK)])

        for cp in scats:
            cp.wait()

    return k(data, scores, g, struct_src, struct_dst)


def kernel(data, structure, W, b):
    scores2d = _scores_tc(data, W, b)
    g2d = _select_tc(scores2d.reshape(ROWS128, 128))
    pooled, ps_flat = _sc_pool(
        data, scores2d.reshape(NPAD), g2d.reshape(NPAD),
        structure[0], structure[1])
    return pooled[:K_KEEP], ps_flat.reshape(2, N_EDGES)


# speedup vs baseline: 50.9050x; 1.1762x over previous
import functools

import jax
import jax.numpy as jnp
from jax import lax
from jax.experimental import pallas as pl
from jax.experimental.pallas import tpu as pltpu
from jax.experimental.pallas import tpu_sc as plsc

N_NODES = 10000
D_FEAT = 256
N_EDGES = 160000
NPAD = 10240
ROWS128 = NPAD // 128
K_KEEP = N_NODES // 2
NTILES = 32
OUT_ROWS = 160
OUT_GROUPS8 = K_KEEP // 8
EDGE_CHUNK = N_EDGES // NTILES
EDGE_STAGE = EDGE_CHUNK + 8
EPAD = N_EDGES + 16


def _mv_body(d_ref, w_ref, b_ref, o_ref):
    o_ref[...] = (
        jnp.dot(d_ref[...], w_ref[...], preferred_element_type=jnp.float32)
        + b_ref[0, 0]
    )


def _sel_body(s_ref, g_ref):
    s = s_ref[...]
    u = lax.bitcast_convert_type(s, jnp.uint32)
    key = jnp.where(u >= jnp.uint32(0x80000000), ~u, u | jnp.uint32(0x80000000))
    row = lax.broadcasted_iota(jnp.int32, (ROWS128, 128), 0)
    lane = lax.broadcasted_iota(jnp.int32, (ROWS128, 128), 1)
    flat = row * 128 + lane
    key = jnp.where(flat < N_NODES, key, jnp.uint32(0))

    def bs(i, ans):
        sh = jnp.uint32(31) - i.astype(jnp.uint32)
        cand = ans | (jnp.uint32(1) << sh)
        cnt = jnp.sum((key >= cand).astype(jnp.int32))
        return lax.select(cnt >= K_KEEP, cand, ans)

    thr = lax.fori_loop(0, 32, bs, jnp.uint32(0))
    cnt_gt = jnp.sum((key > thr).astype(jnp.int32))
    need = (K_KEEP - cnt_gt).astype(jnp.float32)
    eq = key == thr
    upper = (
        lax.broadcasted_iota(jnp.int32, (128, 128), 0)
        <= lax.broadcasted_iota(jnp.int32, (128, 128), 1)
    ).astype(jnp.float32)
    lstrict = (
        lax.broadcasted_iota(jnp.int32, (ROWS128, ROWS128), 0)
        > lax.broadcasted_iota(jnp.int32, (ROWS128, ROWS128), 1)
    ).astype(jnp.float32)
    ef = eq.astype(jnp.float32)
    incl_e = jnp.dot(ef, upper, preferred_element_type=jnp.float32)
    off_e = jnp.dot(lstrict, incl_e[:, 127:128], preferred_element_type=jnp.float32)
    excl_rank = incl_e - ef + off_e
    kept = (key > thr) | (eq & (excl_rank < need))
    kf = kept.astype(jnp.float32)
    incl_k = jnp.dot(kf, upper, preferred_element_type=jnp.float32)
    off_k = jnp.dot(lstrict, incl_k[:, 127:128], preferred_element_type=jnp.float32)
    new_id = (incl_k + off_k - 1.0).astype(jnp.int32)
    g_ref[...] = jnp.where(kept, new_id, -1)


def _scores_tc(data, W, b):
    return pl.pallas_call(
        _mv_body,
        grid=(8,),
        in_specs=[
            pl.BlockSpec((1280, D_FEAT), lambda i: (i, 0)),
            pl.BlockSpec((D_FEAT, 1), lambda i: (0, 0)),
            pl.BlockSpec((1, 1), lambda i: (0, 0)),
        ],
        out_specs=pl.BlockSpec((1280, 1), lambda i: (i, 0)),
        out_shape=jax.ShapeDtypeStruct((NPAD, 1), jnp.float32),
    )(data, W, b.reshape(1, 1))


def _select_tc(scores2d):
    return pl.pallas_call(
        _sel_body,
        out_shape=jax.ShapeDtypeStruct((ROWS128, 128), jnp.int32),
    )(scores2d)


def _sc_pool(data, scores, g, struct_src, struct_dst):
    mesh = plsc.VectorSubcoreMesh(core_axis_name="c", subcore_axis_name="s")

    @functools.partial(
        pl.kernel,
        mesh=mesh,
        compiler_params=pltpu.CompilerParams(needs_layout_passes=False),
        out_type=[
            jax.ShapeDtypeStruct((K_KEEP, D_FEAT), jnp.float32),
            jax.ShapeDtypeStruct((2 * N_EDGES,), jnp.int32),
        ],
        scratch_types=[
            pltpu.VMEM((OUT_ROWS, D_FEAT), jnp.float32),
            pltpu.VMEM((NPAD,), jnp.int32),
            pltpu.VMEM((NPAD,), jnp.float32),
            pltpu.VMEM((OUT_ROWS + 16,), jnp.int32),
            pltpu.VMEM((OUT_ROWS,), jnp.int32),
            pltpu.VMEM((EDGE_STAGE,), jnp.int32),
            pltpu.VMEM((EDGE_STAGE,), jnp.int32),
            pltpu.VMEM((EDGE_STAGE,), jnp.int32),
            pltpu.VMEM((EDGE_STAGE,), jnp.int32),
            pltpu.SemaphoreType.DMA,
            pltpu.SemaphoreType.DMA,
        ],
    )
    def k(data_hbm, scores_hbm, g_hbm, src_hbm, dst_hbm,
          pooled_hbm, ps_hbm,
          dbuf, gt, scb, kbuf, kidx, esrc, edst, eo0, eo1, sem_in, sem_e):
        wid = lax.axis_index("s") * 2 + lax.axis_index("c")
        ob0 = ((wid * OUT_GROUPS8) // NTILES) * 8
        e0 = wid * EDGE_CHUNK
        def _estage(n):
            return (
                pltpu.make_async_copy(
                    src_hbm.at[pl.ds(e0, n)], esrc.at[pl.ds(0, n)], sem_e),
                pltpu.make_async_copy(
                    dst_hbm.at[pl.ds(e0, n)], edst.at[pl.ds(0, n)], sem_e),
            )

        @pl.when(wid < NTILES - 1)
        def _():
            for cp in _estage(EDGE_STAGE):
                cp.start()

        @pl.when(wid == NTILES - 1)
        def _():
            for cp in _estage(EDGE_CHUNK):
                cp.start()
        pltpu.sync_copy(g_hbm, gt)
        pltpu.sync_copy(scores_hbm, scb)

        iota16 = lax.iota(jnp.int32, 16)
        hi = ob0 + OUT_ROWS

        def scan_body(i, cnt):
            gv = gt[pl.ds(i * 16, 16)]
            m = (gv >= ob0) & (gv < hi)
            ids = lax.broadcast(i * 16, (16,)) + iota16
            plsc.store_compressed(kbuf.at[pl.ds(cnt, 16)], ids, mask=m)
            return cnt + plsc.all_reduce_population_count(m)[0]

        lax.fori_loop(0, NPAD // 16, scan_body, 0)
        for j in range(OUT_ROWS // 16):
            kidx[pl.ds(j * 16, 16)] = kbuf[pl.ds(j * 16, 16)]

        pltpu.make_async_copy(data_hbm.at[kidx], dbuf, sem_in).start()

        @pl.when(wid < NTILES - 1)
        def _():
            for cp in _estage(EDGE_STAGE):
                cp.wait()

        @pl.when(wid == NTILES - 1)
        def _():
            for cp in _estage(EDGE_CHUNK):
                cp.wait()

        def ebody(i, carry):
            sv = esrc[pl.ds(i * 16, 16)]
            dv = edst[pl.ds(i * 16, 16)]
            sv = jnp.minimum(jnp.maximum(sv, 0), NPAD - 1)
            dv = jnp.minimum(jnp.maximum(dv, 0), NPAD - 1)
            a = plsc.load_gather(gt, [sv])
            bb = plsc.load_gather(gt, [dv])
            m = (a >= 0) & (bb >= 0)
            eo0[pl.ds(i * 16, 16)] = jnp.where(m, a, -1)
            eo1[pl.ds(i * 16, 16)] = jnp.where(m, bb, -1)
            return carry

        lax.fori_loop(0, EDGE_STAGE // 16, ebody, 0)

        @pl.when(wid < NTILES - 1)
        def _():
            pltpu.sync_copy(eo0.at[pl.ds(0, EDGE_STAGE)],
                            ps_hbm.at[pl.ds(e0, EDGE_STAGE)])
            pltpu.sync_copy(eo1.at[pl.ds(0, EDGE_STAGE)],
                            ps_hbm.at[pl.ds(N_EDGES + e0, EDGE_STAGE)])

        @pl.when(wid == NTILES - 1)
        def _():
            pltpu.sync_copy(eo0.at[pl.ds(0, EDGE_CHUNK)],
                            ps_hbm.at[pl.ds(e0, EDGE_CHUNK)])
            pltpu.sync_copy(eo1.at[pl.ds(0, EDGE_CHUNK)],
                            ps_hbm.at[pl.ds(N_EDGES + e0, EDGE_CHUNK)])

        pltpu.make_async_copy(data_hbm.at[kidx], dbuf, sem_in).wait()

        def grp_body(j, carry):
            kv = kidx[pl.ds(j * 16, 16)]
            svec = plsc.load_gather(scb, [kv])
            for r in range(16):
                vv = lax.broadcast(svec[r], (16,))
                row = j * 16 + r
                for c in range(D_FEAT // 16):
                    dbuf[row, pl.ds(c * 16, 16)] = (
                        dbuf[row, pl.ds(c * 16, 16)] * vv)
            return carry

        lax.fori_loop(0, OUT_ROWS // 16, grp_body, 0)
        pltpu.sync_copy(dbuf, pooled_hbm.at[pl.ds(ob0, OUT_ROWS), :])

    return k(data, scores, g, struct_src, struct_dst)


def kernel(data, structure, W, b):
    scores2d = _scores_tc(data, W, b)
    g2d = _select_tc(scores2d.reshape(ROWS128, 128))
    pooled, ps_flat = _sc_pool(
        data, scores2d.reshape(NPAD), g2d.reshape(NPAD),
        structure[0], structure[1])
    return pooled, ps_flat.reshape(2, N_EDGES)


# speedup vs baseline: 61.4229x; 1.2066x over previous
import functools

import jax
import jax.numpy as jnp
from jax import lax
from jax.experimental import pallas as pl
from jax.experimental.pallas import tpu as pltpu
from jax.experimental.pallas import tpu_sc as plsc

N_NODES = 10000
D_FEAT = 256
N_EDGES = 160000
NPAD = 10240
ROWS128 = NPAD // 128
K_KEEP = N_NODES // 2
NTILES = 32
OUT_ROWS = 160
OUT_GROUPS8 = K_KEEP // 8
EDGE_BIG = 5120
EDGE_SMALL = 4992


def _mv_body(d_ref, w_ref, b_ref, o_ref):
    o_ref[...] = (
        jnp.dot(d_ref[...], w_ref[...], preferred_element_type=jnp.float32)
        + b_ref[0, 0]
    )


def _sel_body(s_ref, g_ref):
    s = s_ref[...]
    u = lax.bitcast_convert_type(s, jnp.uint32)
    key = jnp.where(u >= jnp.uint32(0x80000000), ~u, u | jnp.uint32(0x80000000))
    row = lax.broadcasted_iota(jnp.int32, (ROWS128, 128), 0)
    lane = lax.broadcasted_iota(jnp.int32, (ROWS128, 128), 1)
    flat = row * 128 + lane
    key = jnp.where(flat < N_NODES, key, jnp.uint32(0))

    def bs(i, ans):
        sh = jnp.uint32(31) - i.astype(jnp.uint32)
        cand = ans | (jnp.uint32(1) << sh)
        cnt = jnp.sum((key >= cand).astype(jnp.int32))
        return lax.select(cnt >= K_KEEP, cand, ans)

    thr = lax.fori_loop(0, 32, bs, jnp.uint32(0))
    cnt_gt = jnp.sum((key > thr).astype(jnp.int32))
    need = (K_KEEP - cnt_gt).astype(jnp.float32)
    eq = key == thr
    upper = (
        lax.broadcasted_iota(jnp.int32, (128, 128), 0)
        <= lax.broadcasted_iota(jnp.int32, (128, 128), 1)
    ).astype(jnp.float32)
    lstrict = (
        lax.broadcasted_iota(jnp.int32, (ROWS128, ROWS128), 0)
        > lax.broadcasted_iota(jnp.int32, (ROWS128, ROWS128), 1)
    ).astype(jnp.float32)
    ef = eq.astype(jnp.float32)
    incl_e = jnp.dot(ef, upper, preferred_element_type=jnp.float32)
    off_e = jnp.dot(lstrict, incl_e[:, 127:128], preferred_element_type=jnp.float32)
    excl_rank = incl_e - ef + off_e
    kept = (key > thr) | (eq & (excl_rank < need))
    kf = kept.astype(jnp.float32)
    incl_k = jnp.dot(kf, upper, preferred_element_type=jnp.float32)
    off_k = jnp.dot(lstrict, incl_k[:, 127:128], preferred_element_type=jnp.float32)
    new_id = (incl_k + off_k - 1.0).astype(jnp.int32)
    g_ref[...] = jnp.where(kept, new_id, -1)


def _scores_tc(data, W, b):
    return pl.pallas_call(
        _mv_body,
        grid=(8,),
        in_specs=[
            pl.BlockSpec((1280, D_FEAT), lambda i: (i, 0)),
            pl.BlockSpec((D_FEAT, 1), lambda i: (0, 0)),
            pl.BlockSpec((1, 1), lambda i: (0, 0)),
        ],
        out_specs=pl.BlockSpec((1280, 1), lambda i: (i, 0)),
        out_shape=jax.ShapeDtypeStruct((NPAD, 1), jnp.float32),
    )(data, W, b.reshape(1, 1))


def _select_tc(scores2d):
    return pl.pallas_call(
        _sel_body,
        out_shape=jax.ShapeDtypeStruct((ROWS128, 128), jnp.int32),
    )(scores2d)


def _sc_pool(data, scores, g, structure):
    mesh = plsc.VectorSubcoreMesh(core_axis_name="c", subcore_axis_name="s")

    @functools.partial(
        pl.kernel,
        mesh=mesh,
        compiler_params=pltpu.CompilerParams(needs_layout_passes=False),
        out_type=[
            jax.ShapeDtypeStruct((K_KEEP, D_FEAT), jnp.float32),
            jax.ShapeDtypeStruct((2, N_EDGES), jnp.int32),
        ],
        scratch_types=[
            pltpu.VMEM((OUT_ROWS, D_FEAT), jnp.float32),
            pltpu.VMEM((NPAD,), jnp.int32),
            pltpu.VMEM((NPAD,), jnp.float32),
            pltpu.VMEM((OUT_ROWS + 16,), jnp.int32),
            pltpu.VMEM((OUT_ROWS,), jnp.int32),
            pltpu.VMEM((2, EDGE_BIG), jnp.int32),
            pltpu.VMEM((2, EDGE_BIG), jnp.int32),
            pltpu.SemaphoreType.DMA,
            pltpu.SemaphoreType.DMA,
        ],
    )
    def k(data_hbm, scores_hbm, g_hbm, st_hbm,
          pooled_hbm, ps_hbm,
          dbuf, gt, scb, kbuf, kidx, ebuf, ebo, sem_in, sem_e):
        wid = lax.axis_index("s") * 2 + lax.axis_index("c")
        ob0 = ((wid * OUT_GROUPS8) // NTILES) * 8
        e0 = jnp.where(wid < 2, wid * EDGE_BIG,
                       2 * EDGE_BIG + (wid - 2) * EDGE_SMALL)

        @pl.when(wid < 2)
        def _():
            pltpu.make_async_copy(
                st_hbm.at[:, pl.ds(e0, EDGE_BIG)], ebuf, sem_e).start()

        @pl.when(wid >= 2)
        def _():
            pltpu.make_async_copy(
                st_hbm.at[:, pl.ds(e0, EDGE_SMALL)],
                ebuf.at[:, pl.ds(0, EDGE_SMALL)], sem_e).start()
        pltpu.sync_copy(g_hbm, gt)
        pltpu.sync_copy(scores_hbm, scb)

        iota16 = lax.iota(jnp.int32, 16)
        hi = ob0 + OUT_ROWS

        def scan_body(i, cnt):
            gv = gt[pl.ds(i * 16, 16)]
            m = (gv >= ob0) & (gv < hi)
            ids = lax.broadcast(i * 16, (16,)) + iota16
            plsc.store_compressed(kbuf.at[pl.ds(cnt, 16)], ids, mask=m)
            return cnt + plsc.all_reduce_population_count(m)[0]

        lax.fori_loop(0, NPAD // 16, scan_body, 0)
        for j in range(OUT_ROWS // 16):
            kidx[pl.ds(j * 16, 16)] = kbuf[pl.ds(j * 16, 16)]

        pltpu.make_async_copy(data_hbm.at[kidx], dbuf, sem_in).start()

        @pl.when(wid < 2)
        def _():
            pltpu.make_async_copy(
                st_hbm.at[:, pl.ds(e0, EDGE_BIG)], ebuf, sem_e).wait()

        @pl.when(wid >= 2)
        def _():
            pltpu.make_async_copy(
                st_hbm.at[:, pl.ds(e0, EDGE_SMALL)],
                ebuf.at[:, pl.ds(0, EDGE_SMALL)], sem_e).wait()

        nvec = jnp.where(wid < 2, EDGE_BIG // 16, EDGE_SMALL // 16)

        def ebody(i, carry):
            sv = ebuf[0, pl.ds(i * 16, 16)]
            dv = ebuf[1, pl.ds(i * 16, 16)]
            a = plsc.load_gather(gt, [sv])
            bb = plsc.load_gather(gt, [dv])
            m = (a >= 0) & (bb >= 0)
            ebo[0, pl.ds(i * 16, 16)] = jnp.where(m, a, -1)
            ebo[1, pl.ds(i * 16, 16)] = jnp.where(m, bb, -1)
            return carry

        lax.fori_loop(0, nvec, ebody, 0)

        @pl.when(wid < 2)
        def _():
            pltpu.sync_copy(ebo, ps_hbm.at[:, pl.ds(e0, EDGE_BIG)])

        @pl.when(wid >= 2)
        def _():
            pltpu.sync_copy(ebo.at[:, pl.ds(0, EDGE_SMALL)],
                            ps_hbm.at[:, pl.ds(e0, EDGE_SMALL)])

        pltpu.make_async_copy(data_hbm.at[kidx], dbuf, sem_in).wait()

        def grp_body(j, carry):
            kv = kidx[pl.ds(j * 16, 16)]
            svec = plsc.load_gather(scb, [kv])
            for r in range(16):
                vv = lax.broadcast(svec[r], (16,))
                row = j * 16 + r
                for c in range(D_FEAT // 16):
                    dbuf[row, pl.ds(c * 16, 16)] = (
                        dbuf[row, pl.ds(c * 16, 16)] * vv)
            return carry

        lax.fori_loop(0, OUT_ROWS // 16, grp_body, 0)
        pltpu.sync_copy(dbuf, pooled_hbm.at[pl.ds(ob0, OUT_ROWS), :])

    return k(data, scores, g, structure)


def kernel(data, structure, W, b):
    scores2d = _scores_tc(data, W, b)
    g2d = _select_tc(scores2d.reshape(ROWS128, 128))
    pooled, ps = _sc_pool(
        data, scores2d.reshape(NPAD), g2d.reshape(NPAD), structure)
    return pooled, ps


# speedup vs baseline: 68.8411x; 1.1208x over previous
import functools

import jax
import jax.numpy as jnp
from jax import lax
from jax.experimental import pallas as pl
from jax.experimental.pallas import tpu as pltpu
from jax.experimental.pallas import tpu_sc as plsc

N_NODES = 10000
D_FEAT = 256
N_EDGES = 160000
NPAD = 10240
ROWS128 = NPAD // 128
K_KEEP = N_NODES // 2
NTILES = 32
OUT_ROWS = 160
OUT_GROUPS8 = K_KEEP // 8
EDGE_BIG = 5120
EDGE_SMALL = 4992


def _mv_body(d_ref, w_ref, b_ref, o_ref):
    o_ref[...] = (
        jnp.dot(d_ref[...], w_ref[...], preferred_element_type=jnp.float32)
        + b_ref[0, 0]
    )


def _sel_body(s_ref, g_ref, c_ref):
    s = s_ref[...]
    u = lax.bitcast_convert_type(s, jnp.uint32)
    key = jnp.where(u >= jnp.uint32(0x80000000), ~u, u | jnp.uint32(0x80000000))
    row = lax.broadcasted_iota(jnp.int32, (ROWS128, 128), 0)
    lane = lax.broadcasted_iota(jnp.int32, (ROWS128, 128), 1)
    flat = row * 128 + lane
    key = jnp.where(flat < N_NODES, key, jnp.uint32(0))

    def bs(i, ans):
        sh = jnp.uint32(31) - i.astype(jnp.uint32)
        cand = ans | (jnp.uint32(1) << sh)
        cnt = jnp.sum((key >= cand).astype(jnp.int32))
        return lax.select(cnt >= K_KEEP, cand, ans)

    thr = lax.fori_loop(0, 32, bs, jnp.uint32(0))
    cnt_gt = jnp.sum((key > thr).astype(jnp.int32))
    need = (K_KEEP - cnt_gt).astype(jnp.float32)
    eq = key == thr
    upper = (
        lax.broadcasted_iota(jnp.int32, (128, 128), 0)
        <= lax.broadcasted_iota(jnp.int32, (128, 128), 1)
    ).astype(jnp.float32)
    lstrict = (
        lax.broadcasted_iota(jnp.int32, (ROWS128, ROWS128), 0)
        > lax.broadcasted_iota(jnp.int32, (ROWS128, ROWS128), 1)
    ).astype(jnp.float32)
    ef = eq.astype(jnp.float32)
    incl_e = jnp.dot(ef, upper, preferred_element_type=jnp.float32)
    off_e = jnp.dot(lstrict, incl_e[:, 127:128], preferred_element_type=jnp.float32)
    excl_rank = incl_e - ef + off_e
    kept = (key > thr) | (eq & (excl_rank < need))
    kf = kept.astype(jnp.float32)
    incl_k = jnp.dot(kf, upper, preferred_element_type=jnp.float32)
    off_k = jnp.dot(lstrict, incl_k[:, 127:128], preferred_element_type=jnp.float32)
    cnt = (incl_k + off_k).astype(jnp.int32)
    g_ref[...] = jnp.where(kept, cnt - 1, -1)
    c_ref[...] = cnt


def _scores_tc(data, W, b):
    return pl.pallas_call(
        _mv_body,
        grid=(8,),
        in_specs=[
            pl.BlockSpec((1280, D_FEAT), lambda i: (i, 0)),
            pl.BlockSpec((D_FEAT, 1), lambda i: (0, 0)),
            pl.BlockSpec((1, 1), lambda i: (0, 0)),
        ],
        out_specs=pl.BlockSpec((1280, 1), lambda i: (i, 0)),
        out_shape=jax.ShapeDtypeStruct((NPAD, 1), jnp.float32),
    )(data, W, b.reshape(1, 1))


def _select_tc(scores2d):
    return pl.pallas_call(
        _sel_body,
        out_shape=[
            jax.ShapeDtypeStruct((ROWS128, 128), jnp.int32),
            jax.ShapeDtypeStruct((ROWS128, 128), jnp.int32),
        ],
    )(scores2d)


def _sc_pool(data, scores, g, cnt, structure):
    mesh = plsc.VectorSubcoreMesh(core_axis_name="c", subcore_axis_name="s")

    @functools.partial(
        pl.kernel,
        mesh=mesh,
        compiler_params=pltpu.CompilerParams(needs_layout_passes=False),
        out_type=[
            jax.ShapeDtypeStruct((K_KEEP, D_FEAT), jnp.float32),
            jax.ShapeDtypeStruct((2, N_EDGES), jnp.int32),
        ],
        scratch_types=[
            pltpu.VMEM((OUT_ROWS, D_FEAT), jnp.float32),
            pltpu.VMEM((NPAD,), jnp.int32),
            pltpu.VMEM((NPAD,), jnp.float32),
            pltpu.VMEM((NPAD,), jnp.int32),
            pltpu.VMEM((OUT_ROWS + 16,), jnp.int32),
            pltpu.VMEM((OUT_ROWS,), jnp.int32),
            pltpu.VMEM((2, EDGE_BIG), jnp.int32),
            pltpu.VMEM((2, EDGE_BIG), jnp.int32),
            pltpu.SemaphoreType.DMA,
            pltpu.SemaphoreType.DMA,
            pltpu.SemaphoreType.DMA,
            pltpu.SemaphoreType.DMA,
            pltpu.SemaphoreType.DMA,
            pltpu.SemaphoreType.DMA,
        ],
    )
    def k(data_hbm, scores_hbm, g_hbm, cnt_hbm, st_hbm,
          pooled_hbm, ps_hbm,
          dbuf, gt, scb, cb, kbuf, kidx, ebuf, ebo,
          sem_in, sem_e, sem_gt, sem_sc, sem_cnt, sem_eo):
        wid = lax.axis_index("s") * 2 + lax.axis_index("c")
        ob0 = ((wid * OUT_GROUPS8) // NTILES) * 8
        e0 = jnp.where(wid < 2, wid * EDGE_BIG,
                       2 * EDGE_BIG + (wid - 2) * EDGE_SMALL)

        @pl.when(wid < 2)
        def _():
            pltpu.make_async_copy(
                st_hbm.at[:, pl.ds(e0, EDGE_BIG)], ebuf, sem_e).start()

        @pl.when(wid >= 2)
        def _():
            pltpu.make_async_copy(
                st_hbm.at[:, pl.ds(e0, EDGE_SMALL)],
                ebuf.at[:, pl.ds(0, EDGE_SMALL)], sem_e).start()
        cp_gt = pltpu.make_async_copy(g_hbm, gt, sem_gt)
        cp_gt.start()
        cp_sc = pltpu.make_async_copy(scores_hbm, scb, sem_sc)
        cp_sc.start()
        cp_cnt = pltpu.make_async_copy(cnt_hbm, cb, sem_cnt)
        cp_cnt.start()

        cp_cnt.wait()

        def first_geq(target):
            lo = jnp.int32(0)
            for s in (8192, 4096, 2048, 1024, 512, 256, 128, 64, 32, 16,
                      8, 4, 2, 1):
                cand = lo + s
                idx = jnp.minimum(cand - 1, NPAD - 1)
                v = plsc.load_gather(cb, [lax.broadcast(idx, (16,))])[0]
                ok = (cand <= NPAD) & (v < target)
                lo = jnp.where(ok, cand, lo)
            return lo

        hi = ob0 + OUT_ROWS
        blo = first_geq(ob0 + 1) // 16
        bhi = first_geq(hi) // 16 + 1

        iota16 = lax.iota(jnp.int32, 16)
        cp_gt.wait()

        def scan_body(i, c):
            gv = gt[pl.ds(i * 16, 16)]
            m = (gv >= ob0) & (gv < hi)
            ids = lax.broadcast(i * 16, (16,)) + iota16
            plsc.store_compressed(kbuf.at[pl.ds(c, 16)], ids, mask=m)
            return c + plsc.all_reduce_population_count(m)[0]

        lax.fori_loop(blo, bhi, scan_body, 0)
        for j in range(OUT_ROWS // 16):
            kidx[pl.ds(j * 16, 16)] = kbuf[pl.ds(j * 16, 16)]

        pltpu.make_async_copy(data_hbm.at[kidx], dbuf, sem_in).start()

        @pl.when(wid < 2)
        def _():
            pltpu.make_async_copy(
                st_hbm.at[:, pl.ds(e0, EDGE_BIG)], ebuf, sem_e).wait()

        @pl.when(wid >= 2)
        def _():
            pltpu.make_async_copy(
                st_hbm.at[:, pl.ds(e0, EDGE_SMALL)],
                ebuf.at[:, pl.ds(0, EDGE_SMALL)], sem_e).wait()

        nvec = jnp.where(wid < 2, EDGE_BIG // 16, EDGE_SMALL // 16)

        def ebody(i, carry):
            sv = ebuf[0, pl.ds(i * 16, 16)]
            dv = ebuf[1, pl.ds(i * 16, 16)]
            a = plsc.load_gather(gt, [sv])
            bb = plsc.load_gather(gt, [dv])
            m = (a >= 0) & (bb >= 0)
            ebo[0, pl.ds(i * 16, 16)] = jnp.where(m, a, -1)
            ebo[1, pl.ds(i * 16, 16)] = jnp.where(m, bb, -1)
            return carry

        lax.fori_loop(0, nvec, ebody, 0)

        @pl.when(wid < 2)
        def _():
            pltpu.make_async_copy(
                ebo, ps_hbm.at[:, pl.ds(e0, EDGE_BIG)], sem_eo).start()

        @pl.when(wid >= 2)
        def _():
            pltpu.make_async_copy(
                ebo.at[:, pl.ds(0, EDGE_SMALL)],
                ps_hbm.at[:, pl.ds(e0, EDGE_SMALL)], sem_eo).start()

        pltpu.make_async_copy(data_hbm.at[kidx], dbuf, sem_in).wait()
        cp_sc.wait()

        def grp_body(j, carry):
            kv = kidx[pl.ds(j * 16, 16)]
            svec = plsc.load_gather(scb, [kv])
            for r in range(16):
                vv = lax.broadcast(svec[r], (16,))
                row = j * 16 + r
                for c in range(D_FEAT // 16):
                    dbuf[row, pl.ds(c * 16, 16)] = (
                        dbuf[row, pl.ds(c * 16, 16)] * vv)
            return carry

        lax.fori_loop(0, OUT_ROWS // 16, grp_body, 0)
        pltpu.sync_copy(dbuf, pooled_hbm.at[pl.ds(ob0, OUT_ROWS), :])

        @pl.when(wid < 2)
        def _():
            pltpu.make_async_copy(
                ebo, ps_hbm.at[:, pl.ds(e0, EDGE_BIG)], sem_eo).wait()

        @pl.when(wid >= 2)
        def _():
            pltpu.make_async_copy(
                ebo.at[:, pl.ds(0, EDGE_SMALL)],
                ps_hbm.at[:, pl.ds(e0, EDGE_SMALL)], sem_eo).wait()

    return k(data, scores, g, cnt, structure)


def kernel(data, structure, W, b):
    scores2d = _scores_tc(data, W, b)
    g2d, cnt2d = _select_tc(scores2d.reshape(ROWS128, 128))
    pooled, ps = _sc_pool(
        data, scores2d.reshape(NPAD), g2d.reshape(NPAD),
        cnt2d.reshape(NPAD), structure)
    return pooled, ps


# speedup vs baseline: 78.6552x; 1.1426x over previous
import functools

import jax
import jax.numpy as jnp
from jax import lax
from jax.experimental import pallas as pl
from jax.experimental.pallas import tpu as pltpu
from jax.experimental.pallas import tpu_sc as plsc

N_NODES = 10000
D_FEAT = 256
N_EDGES = 160000
NPAD = 10240
ROWS128 = NPAD // 128
K_KEEP = N_NODES // 2
NTILES = 32
OUT_ROWS = 160
OUT_GROUPS8 = K_KEEP // 8
EDGE_BIG = 5120
EDGE_SMALL = 4992


def _mv_body(d_ref, w_ref, b_ref, o_ref):
    wt = w_ref[...]
    for rr in range(8):
        blk = d_ref[pl.ds(rr * 128, 128), :]
        o_ref[pl.ds(rr, 1), :] = (
            lax.dot_general(wt, blk, (((1,), (1,)), ((), ())),
                            preferred_element_type=jnp.float32)
            + b_ref[0, 0]
        )


def _sel_body(s_ref, g_ref, c_ref):
    s = s_ref[...]
    u = lax.bitcast_convert_type(s, jnp.uint32)
    key = jnp.where(u >= jnp.uint32(0x80000000), ~u, u | jnp.uint32(0x80000000))
    row = lax.broadcasted_iota(jnp.int32, (ROWS128, 128), 0)
    lane = lax.broadcasted_iota(jnp.int32, (ROWS128, 128), 1)
    flat = row * 128 + lane
    key = jnp.where(flat < N_NODES, key, jnp.uint32(0))

    def bs(i, ans):
        sh = jnp.uint32(31) - i.astype(jnp.uint32)
        cand = ans | (jnp.uint32(1) << sh)
        cnt = jnp.sum((key >= cand).astype(jnp.int32))
        return lax.select(cnt >= K_KEEP, cand, ans)

    thr = lax.fori_loop(0, 32, bs, jnp.uint32(0))
    cnt_gt = jnp.sum((key > thr).astype(jnp.int32))
    need = (K_KEEP - cnt_gt).astype(jnp.float32)
    eq = key == thr
    upper = (
        lax.broadcasted_iota(jnp.int32, (128, 128), 0)
        <= lax.broadcasted_iota(jnp.int32, (128, 128), 1)
    ).astype(jnp.float32)
    lstrict = (
        lax.broadcasted_iota(jnp.int32, (ROWS128, ROWS128), 0)
        > lax.broadcasted_iota(jnp.int32, (ROWS128, ROWS128), 1)
    ).astype(jnp.float32)
    ef = eq.astype(jnp.float32)
    incl_e = jnp.dot(ef, upper, preferred_element_type=jnp.float32)
    off_e = jnp.dot(lstrict, incl_e[:, 127:128], preferred_element_type=jnp.float32)
    excl_rank = incl_e - ef + off_e
    kept = (key > thr) | (eq & (excl_rank < need))
    kf = kept.astype(jnp.float32)
    incl_k = jnp.dot(kf, upper, preferred_element_type=jnp.float32)
    off_k = jnp.dot(lstrict, incl_k[:, 127:128], preferred_element_type=jnp.float32)
    cnt = (incl_k + off_k).astype(jnp.int32)
    g_ref[...] = jnp.where(kept, cnt - 1, -1)
    c_ref[...] = cnt


def _scores_tc(data, W, b):
    return pl.pallas_call(
        _mv_body,
        grid=(10,),
        in_specs=[
            pl.BlockSpec((1024, D_FEAT), lambda i: (i, 0)),
            pl.BlockSpec((1, D_FEAT), lambda i: (0, 0)),
            pl.BlockSpec((1, 1), lambda i: (0, 0)),
        ],
        out_specs=pl.BlockSpec((8, 128), lambda i: (i, 0)),
        out_shape=jax.ShapeDtypeStruct((ROWS128, 128), jnp.float32),
    )(data, W.reshape(1, D_FEAT), b.reshape(1, 1))


def _select_tc(scores2d):
    return pl.pallas_call(
        _sel_body,
        out_shape=[
            jax.ShapeDtypeStruct((ROWS128, 128), jnp.int32),
            jax.ShapeDtypeStruct((ROWS128, 128), jnp.int32),
        ],
    )(scores2d)


def _sc_pool(data, scores, g, cnt, structure):
    mesh = plsc.VectorSubcoreMesh(core_axis_name="c", subcore_axis_name="s")

    @functools.partial(
        pl.kernel,
        mesh=mesh,
        compiler_params=pltpu.CompilerParams(needs_layout_passes=False),
        out_type=[
            jax.ShapeDtypeStruct((K_KEEP, D_FEAT), jnp.float32),
            jax.ShapeDtypeStruct((2, N_EDGES), jnp.int32),
        ],
        scratch_types=[
            pltpu.VMEM((OUT_ROWS, D_FEAT), jnp.float32),
            pltpu.VMEM((NPAD,), jnp.int32),
            pltpu.VMEM((NPAD,), jnp.float32),
            pltpu.VMEM((NPAD,), jnp.int32),
            pltpu.VMEM((OUT_ROWS + 16,), jnp.int32),
            pltpu.VMEM((OUT_ROWS,), jnp.int32),
            pltpu.VMEM((2, EDGE_BIG), jnp.int32),
            pltpu.VMEM((2, EDGE_BIG), jnp.int32),
            pltpu.SemaphoreType.DMA,
            pltpu.SemaphoreType.DMA,
            pltpu.SemaphoreType.DMA,
            pltpu.SemaphoreType.DMA,
            pltpu.SemaphoreType.DMA,
            pltpu.SemaphoreType.DMA,
        ],
    )
    def k(data_hbm, scores_hbm, g_hbm, cnt_hbm, st_hbm,
          pooled_hbm, ps_hbm,
          dbuf, gt, scb, cb, kbuf, kidx, ebuf, ebo,
          sem_in, sem_e, sem_gt, sem_sc, sem_cnt, sem_eo):
        wid = lax.axis_index("s") * 2 + lax.axis_index("c")
        ob0 = ((wid * OUT_GROUPS8) // NTILES) * 8
        e0 = jnp.where(wid < 2, wid * EDGE_BIG,
                       2 * EDGE_BIG + (wid - 2) * EDGE_SMALL)

        @pl.when(wid < 2)
        def _():
            pltpu.make_async_copy(
                st_hbm.at[:, pl.ds(e0, EDGE_BIG)], ebuf, sem_e).start()

        @pl.when(wid >= 2)
        def _():
            pltpu.make_async_copy(
                st_hbm.at[:, pl.ds(e0, EDGE_SMALL)],
                ebuf.at[:, pl.ds(0, EDGE_SMALL)], sem_e).start()
        cp_gt = pltpu.make_async_copy(g_hbm, gt, sem_gt)
        cp_gt.start()
        cp_sc = pltpu.make_async_copy(scores_hbm, scb, sem_sc)
        cp_sc.start()
        cp_cnt = pltpu.make_async_copy(cnt_hbm, cb, sem_cnt)
        cp_cnt.start()

        cp_cnt.wait()

        def first_geq(target):
            lo = jnp.int32(0)
            for s in (8192, 4096, 2048, 1024, 512, 256, 128, 64, 32, 16,
                      8, 4, 2, 1):
                cand = lo + s
                idx = jnp.minimum(cand - 1, NPAD - 1)
                v = plsc.load_gather(cb, [lax.broadcast(idx, (16,))])[0]
                ok = (cand <= NPAD) & (v < target)
                lo = jnp.where(ok, cand, lo)
            return lo

        hi = ob0 + OUT_ROWS
        blo = first_geq(ob0 + 1) // 16
        bhi = first_geq(hi) // 16 + 1

        iota16 = lax.iota(jnp.int32, 16)
        cp_gt.wait()

        def scan_body(i, c):
            gv = gt[pl.ds(i * 16, 16)]
            m = (gv >= ob0) & (gv < hi)
            ids = lax.broadcast(i * 16, (16,)) + iota16
            plsc.store_compressed(kbuf.at[pl.ds(c, 16)], ids, mask=m)
            return c + plsc.all_reduce_population_count(m)[0]

        lax.fori_loop(blo, bhi, scan_body, 0)
        for j in range(OUT_ROWS // 16):
            kidx[pl.ds(j * 16, 16)] = kbuf[pl.ds(j * 16, 16)]

        pltpu.make_async_copy(data_hbm.at[kidx], dbuf, sem_in).start()

        @pl.when(wid < 2)
        def _():
            pltpu.make_async_copy(
                st_hbm.at[:, pl.ds(e0, EDGE_BIG)], ebuf, sem_e).wait()

        @pl.when(wid >= 2)
        def _():
            pltpu.make_async_copy(
                st_hbm.at[:, pl.ds(e0, EDGE_SMALL)],
                ebuf.at[:, pl.ds(0, EDGE_SMALL)], sem_e).wait()

        nvec = jnp.where(wid < 2, EDGE_BIG // 16, EDGE_SMALL // 16)

        def ebody(i, carry):
            sv = ebuf[0, pl.ds(i * 16, 16)]
            dv = ebuf[1, pl.ds(i * 16, 16)]
            a = plsc.load_gather(gt, [sv])
            bb = plsc.load_gather(gt, [dv])
            m = (a >= 0) & (bb >= 0)
            ebo[0, pl.ds(i * 16, 16)] = jnp.where(m, a, -1)
            ebo[1, pl.ds(i * 16, 16)] = jnp.where(m, bb, -1)
            return carry

        lax.fori_loop(0, nvec, ebody, 0)

        @pl.when(wid < 2)
        def _():
            pltpu.make_async_copy(
                ebo, ps_hbm.at[:, pl.ds(e0, EDGE_BIG)], sem_eo).start()

        @pl.when(wid >= 2)
        def _():
            pltpu.make_async_copy(
                ebo.at[:, pl.ds(0, EDGE_SMALL)],
                ps_hbm.at[:, pl.ds(e0, EDGE_SMALL)], sem_eo).start()

        pltpu.make_async_copy(data_hbm.at[kidx], dbuf, sem_in).wait()
        cp_sc.wait()

        def grp_body(j, carry):
            kv = kidx[pl.ds(j * 16, 16)]
            svec = plsc.load_gather(scb, [kv])
            for r in range(16):
                vv = lax.broadcast(svec[r], (16,))
                row = j * 16 + r
                for c in range(D_FEAT // 16):
                    dbuf[row, pl.ds(c * 16, 16)] = (
                        dbuf[row, pl.ds(c * 16, 16)] * vv)
            return carry

        lax.fori_loop(0, OUT_ROWS // 16, grp_body, 0)
        pltpu.sync_copy(dbuf, pooled_hbm.at[pl.ds(ob0, OUT_ROWS), :])

        @pl.when(wid < 2)
        def _():
            pltpu.make_async_copy(
                ebo, ps_hbm.at[:, pl.ds(e0, EDGE_BIG)], sem_eo).wait()

        @pl.when(wid >= 2)
        def _():
            pltpu.make_async_copy(
                ebo.at[:, pl.ds(0, EDGE_SMALL)],
                ps_hbm.at[:, pl.ds(e0, EDGE_SMALL)], sem_eo).wait()

    return k(data, scores, g, cnt, structure)


def kernel(data, structure, W, b):
    scores2d = _scores_tc(data, W, b)
    g2d, cnt2d = _select_tc(scores2d)
    pooled, ps = _sc_pool(
        data, scores2d.reshape(NPAD), g2d.reshape(NPAD),
        cnt2d.reshape(NPAD), structure)
    return pooled, ps


# speedup vs baseline: 81.3032x; 1.0337x over previous
import functools

import jax
import jax.numpy as jnp
from jax import lax
from jax.experimental import pallas as pl
from jax.experimental.pallas import tpu as pltpu
from jax.experimental.pallas import tpu_sc as plsc

N_NODES = 10000
D_FEAT = 256
N_EDGES = 160000
NPAD = 10240
ROWS128 = NPAD // 128
K_KEEP = N_NODES // 2
NTILES = 32
OUT_ROWS = 160
OUT_GROUPS8 = K_KEEP // 8
EDGE_BIG = 5120
EDGE_SMALL = 4992


def _mv_sel_body(d_ref, w_ref, b_ref, s_ref, g_ref, c_ref, sb_ref):
    i = pl.program_id(0)
    wt = w_ref[...]
    for rr in range(8):
        blk = d_ref[pl.ds(rr * 128, 128), :]
        row = (
            lax.dot_general(wt, blk, (((1,), (1,)), ((), ())),
                            preferred_element_type=jnp.float32)
            + b_ref[0, 0]
        )
        s_ref[pl.ds(rr, 1), :] = row
        sb_ref[pl.ds(i * 8 + rr, 1), :] = row

    @pl.when(i == 9)
    def _():
        _sel_body(sb_ref, g_ref, c_ref)


def _sel_body(s_ref, g_ref, c_ref):
    s = s_ref[...]
    u = lax.bitcast_convert_type(s, jnp.uint32)
    key = jnp.where(u >= jnp.uint32(0x80000000), ~u, u | jnp.uint32(0x80000000))
    row = lax.broadcasted_iota(jnp.int32, (ROWS128, 128), 0)
    lane = lax.broadcasted_iota(jnp.int32, (ROWS128, 128), 1)
    flat = row * 128 + lane
    key = jnp.where(flat < N_NODES, key, jnp.uint32(0))

    def bs(i, ans):
        sh = jnp.uint32(31) - i.astype(jnp.uint32)
        cand = ans | (jnp.uint32(1) << sh)
        cnt = jnp.sum((key >= cand).astype(jnp.int32))
        return lax.select(cnt >= K_KEEP, cand, ans)

    thr = lax.fori_loop(0, 32, bs, jnp.uint32(0))
    cnt_gt = jnp.sum((key > thr).astype(jnp.int32))
    need = (K_KEEP - cnt_gt).astype(jnp.float32)
    eq = key == thr
    upper = (
        lax.broadcasted_iota(jnp.int32, (128, 128), 0)
        <= lax.broadcasted_iota(jnp.int32, (128, 128), 1)
    ).astype(jnp.float32)
    lstrict = (
        lax.broadcasted_iota(jnp.int32, (ROWS128, ROWS128), 0)
        > lax.broadcasted_iota(jnp.int32, (ROWS128, ROWS128), 1)
    ).astype(jnp.float32)
    ef = eq.astype(jnp.float32)
    incl_e = jnp.dot(ef, upper, preferred_element_type=jnp.float32)
    off_e = jnp.dot(lstrict, incl_e[:, 127:128], preferred_element_type=jnp.float32)
    excl_rank = incl_e - ef + off_e
    kept = (key > thr) | (eq & (excl_rank < need))
    kf = kept.astype(jnp.float32)
    incl_k = jnp.dot(kf, upper, preferred_element_type=jnp.float32)
    off_k = jnp.dot(lstrict, incl_k[:, 127:128], preferred_element_type=jnp.float32)
    cnt = (incl_k + off_k).astype(jnp.int32)
    g_ref[...] = jnp.where(kept, cnt - 1, -1)
    c_ref[...] = cnt


def _scores_select_tc(data, W, b):
    return pl.pallas_call(
        _mv_sel_body,
        grid=(10,),
        in_specs=[
            pl.BlockSpec((1024, D_FEAT), lambda i: (i, 0)),
            pl.BlockSpec((1, D_FEAT), lambda i: (0, 0)),
            pl.BlockSpec((1, 1), lambda i: (0, 0)),
        ],
        out_specs=[
            pl.BlockSpec((8, 128), lambda i: (i, 0)),
            pl.BlockSpec((ROWS128, 128), lambda i: (0, 0)),
            pl.BlockSpec((ROWS128, 128), lambda i: (0, 0)),
        ],
        out_shape=[
            jax.ShapeDtypeStruct((ROWS128, 128), jnp.float32),
            jax.ShapeDtypeStruct((ROWS128, 128), jnp.int32),
            jax.ShapeDtypeStruct((ROWS128, 128), jnp.int32),
        ],
        scratch_shapes=[pltpu.VMEM((ROWS128, 128), jnp.float32)],
    )(data, W.reshape(1, D_FEAT), b.reshape(1, 1))


def _sc_pool(data, scores, g, cnt, structure):
    mesh = plsc.VectorSubcoreMesh(core_axis_name="c", subcore_axis_name="s")

    @functools.partial(
        pl.kernel,
        mesh=mesh,
        compiler_params=pltpu.CompilerParams(needs_layout_passes=False),
        out_type=[
            jax.ShapeDtypeStruct((K_KEEP, D_FEAT), jnp.float32),
            jax.ShapeDtypeStruct((2, N_EDGES), jnp.int32),
        ],
        scratch_types=[
            pltpu.VMEM((OUT_ROWS, D_FEAT), jnp.float32),
            pltpu.VMEM((NPAD,), jnp.int32),
            pltpu.VMEM((NPAD,), jnp.float32),
            pltpu.VMEM((NPAD,), jnp.int32),
            pltpu.VMEM((OUT_ROWS + 16,), jnp.int32),
            pltpu.VMEM((OUT_ROWS,), jnp.int32),
            pltpu.VMEM((2, EDGE_BIG), jnp.int32),
            pltpu.VMEM((2, EDGE_BIG), jnp.int32),
            pltpu.SemaphoreType.DMA,
            pltpu.SemaphoreType.DMA,
            pltpu.SemaphoreType.DMA,
            pltpu.SemaphoreType.DMA,
            pltpu.SemaphoreType.DMA,
            pltpu.SemaphoreType.DMA,
        ],
    )
    def k(data_hbm, scores_hbm, g_hbm, cnt_hbm, st_hbm,
          pooled_hbm, ps_hbm,
          dbuf, gt, scb, cb, kbuf, kidx, ebuf, ebo,
          sem_in, sem_e, sem_gt, sem_sc, sem_cnt, sem_eo):
        wid = lax.axis_index("s") * 2 + lax.axis_index("c")
        ob0 = ((wid * OUT_GROUPS8) // NTILES) * 8
        e0 = jnp.where(wid < 2, wid * EDGE_BIG,
                       2 * EDGE_BIG + (wid - 2) * EDGE_SMALL)

        @pl.when(wid < 2)
        def _():
            pltpu.make_async_copy(
                st_hbm.at[:, pl.ds(e0, EDGE_BIG)], ebuf, sem_e).start()

        @pl.when(wid >= 2)
        def _():
            pltpu.make_async_copy(
                st_hbm.at[:, pl.ds(e0, EDGE_SMALL)],
                ebuf.at[:, pl.ds(0, EDGE_SMALL)], sem_e).start()
        cp_gt = pltpu.make_async_copy(g_hbm, gt, sem_gt)
        cp_gt.start()
        cp_sc = pltpu.make_async_copy(scores_hbm, scb, sem_sc)
        cp_sc.start()
        cp_cnt = pltpu.make_async_copy(cnt_hbm, cb, sem_cnt)
        cp_cnt.start()

        cp_cnt.wait()

        def first_geq(target):
            lo = jnp.int32(0)
            for s in (8192, 4096, 2048, 1024, 512, 256, 128, 64, 32, 16,
                      8, 4, 2, 1):
                cand = lo + s
                idx = jnp.minimum(cand - 1, NPAD - 1)
                v = plsc.load_gather(cb, [lax.broadcast(idx, (16,))])[0]
                ok = (cand <= NPAD) & (v < target)
                lo = jnp.where(ok, cand, lo)
            return lo

        hi = ob0 + OUT_ROWS
        blo = first_geq(ob0 + 1) // 16
        bhi = first_geq(hi) // 16 + 1

        iota16 = lax.iota(jnp.int32, 16)
        cp_gt.wait()

        def scan_body(i, c):
            gv = gt[pl.ds(i * 16, 16)]
            m = (gv >= ob0) & (gv < hi)
            ids = lax.broadcast(i * 16, (16,)) + iota16
            plsc.store_compressed(kbuf.at[pl.ds(c, 16)], ids, mask=m)
            return c + plsc.all_reduce_population_count(m)[0]

        lax.fori_loop(blo, bhi, scan_body, 0)
        for j in range(OUT_ROWS // 16):
            kidx[pl.ds(j * 16, 16)] = kbuf[pl.ds(j * 16, 16)]

        pltpu.make_async_copy(data_hbm.at[kidx], dbuf, sem_in).start()

        @pl.when(wid < 2)
        def _():
            pltpu.make_async_copy(
                st_hbm.at[:, pl.ds(e0, EDGE_BIG)], ebuf, sem_e).wait()

        @pl.when(wid >= 2)
        def _():
            pltpu.make_async_copy(
                st_hbm.at[:, pl.ds(e0, EDGE_SMALL)],
                ebuf.at[:, pl.ds(0, EDGE_SMALL)], sem_e).wait()

        nvec = jnp.where(wid < 2, EDGE_BIG // 16, EDGE_SMALL // 16)

        def ebody(i, carry):
            sv = ebuf[0, pl.ds(i * 16, 16)]
            dv = ebuf[1, pl.ds(i * 16, 16)]
            a = plsc.load_gather(gt, [sv])
            bb = plsc.load_gather(gt, [dv])
            m = (a >= 0) & (bb >= 0)
            ebo[0, pl.ds(i * 16, 16)] = jnp.where(m, a, -1)
            ebo[1, pl.ds(i * 16, 16)] = jnp.where(m, bb, -1)
            return carry

        lax.fori_loop(0, nvec, ebody, 0)

        @pl.when(wid < 2)
        def _():
            pltpu.make_async_copy(
                ebo, ps_hbm.at[:, pl.ds(e0, EDGE_BIG)], sem_eo).start()

        @pl.when(wid >= 2)
        def _():
            pltpu.make_async_copy(
                ebo.at[:, pl.ds(0, EDGE_SMALL)],
                ps_hbm.at[:, pl.ds(e0, EDGE_SMALL)], sem_eo).start()

        pltpu.make_async_copy(data_hbm.at[kidx], dbuf, sem_in).wait()
        cp_sc.wait()

        def grp_body(j, carry):
            kv = kidx[pl.ds(j * 16, 16)]
            svec = plsc.load_gather(scb, [kv])
            for r in range(16):
                vv = lax.broadcast(svec[r], (16,))
                row = j * 16 + r
                for c in range(D_FEAT // 16):
                    dbuf[row, pl.ds(c * 16, 16)] = (
                        dbuf[row, pl.ds(c * 16, 16)] * vv)
            return carry

        lax.fori_loop(0, OUT_ROWS // 16, grp_body, 0)
        pltpu.sync_copy(dbuf, pooled_hbm.at[pl.ds(ob0, OUT_ROWS), :])

        @pl.when(wid < 2)
        def _():
            pltpu.make_async_copy(
                ebo, ps_hbm.at[:, pl.ds(e0, EDGE_BIG)], sem_eo).wait()

        @pl.when(wid >= 2)
        def _():
            pltpu.make_async_copy(
                ebo.at[:, pl.ds(0, EDGE_SMALL)],
                ps_hbm.at[:, pl.ds(e0, EDGE_SMALL)], sem_eo).wait()

    return k(data, scores, g, cnt, structure)


def kernel(data, structure, W, b):
    scores2d, g2d, cnt2d = _scores_select_tc(data, W, b)
    pooled, ps = _sc_pool(
        data, scores2d.reshape(NPAD), g2d.reshape(NPAD),
        cnt2d.reshape(NPAD), structure)
    return pooled, ps


# speedup vs baseline: 85.2222x; 1.0482x over previous
import functools

import jax
import jax.numpy as jnp
from jax import lax
from jax.experimental import pallas as pl
from jax.experimental.pallas import tpu as pltpu
from jax.experimental.pallas import tpu_sc as plsc

N_NODES = 10000
D_FEAT = 256
N_EDGES = 160000
NPAD = 10240
ROWS128 = NPAD // 128
K_KEEP = N_NODES // 2
NTILES = 32
OUT_ROWS = 160
OUT_GROUPS8 = K_KEEP // 8
EDGE_BIG = 5120
EDGE_SMALL = 4992


def _mv_sel_body(d_ref, w_ref, b_ref, s_ref, g_ref, c_ref, sb_ref):
    i = pl.program_id(0)
    wt = w_ref[...]
    for rr in range(16):
        blk = d_ref[pl.ds(rr * 128, 128), :]
        row = (
            lax.dot_general(wt, blk, (((1,), (1,)), ((), ())),
                            preferred_element_type=jnp.float32)
            + b_ref[0, 0]
        )
        s_ref[pl.ds(rr, 1), :] = row
        sb_ref[pl.ds(i * 16 + rr, 1), :] = row

    @pl.when(i == 4)
    def _():
        _sel_body(sb_ref, g_ref, c_ref)


def _sel_body(s_ref, g_ref, c_ref):
    s = s_ref[...]
    u = lax.bitcast_convert_type(s, jnp.uint32)
    key = jnp.where(u >= jnp.uint32(0x80000000), ~u, u | jnp.uint32(0x80000000))
    row = lax.broadcasted_iota(jnp.int32, (ROWS128, 128), 0)
    lane = lax.broadcasted_iota(jnp.int32, (ROWS128, 128), 1)
    flat = row * 128 + lane
    key = jnp.where(flat < N_NODES, key, jnp.uint32(0))

    def bs(i, ans):
        sh = jnp.uint32(31) - i.astype(jnp.uint32)
        cand = ans | (jnp.uint32(1) << sh)
        cnt = jnp.sum((key >= cand).astype(jnp.int32))
        return lax.select(cnt >= K_KEEP, cand, ans)

    thr = lax.fori_loop(0, 32, bs, jnp.uint32(0))
    cnt_gt = jnp.sum((key > thr).astype(jnp.int32))
    need = (K_KEEP - cnt_gt).astype(jnp.float32)
    eq = key == thr
    upper = (
        lax.broadcasted_iota(jnp.int32, (128, 128), 0)
        <= lax.broadcasted_iota(jnp.int32, (128, 128), 1)
    ).astype(jnp.float32)
    lstrict = (
        lax.broadcasted_iota(jnp.int32, (ROWS128, ROWS128), 0)
        > lax.broadcasted_iota(jnp.int32, (ROWS128, ROWS128), 1)
    ).astype(jnp.float32)
    ef = eq.astype(jnp.float32)
    incl_e = jnp.dot(ef, upper, preferred_element_type=jnp.float32)
    off_e = jnp.dot(lstrict, incl_e[:, 127:128], preferred_element_type=jnp.float32)
    excl_rank = incl_e - ef + off_e
    kept = (key > thr) | (eq & (excl_rank < need))
    kf = kept.astype(jnp.float32)
    incl_k = jnp.dot(kf, upper, preferred_element_type=jnp.float32)
    off_k = jnp.dot(lstrict, incl_k[:, 127:128], preferred_element_type=jnp.float32)
    cnt = (incl_k + off_k).astype(jnp.int32)
    g_ref[...] = jnp.where(kept, cnt - 1, -1)
    c_ref[...] = cnt


def _scores_select_tc(data, W, b):
    return pl.pallas_call(
        _mv_sel_body,
        grid=(5,),
        in_specs=[
            pl.BlockSpec((2048, D_FEAT), lambda i: (i, 0)),
            pl.BlockSpec((1, D_FEAT), lambda i: (0, 0)),
            pl.BlockSpec((1, 1), lambda i: (0, 0)),
        ],
        out_specs=[
            pl.BlockSpec((16, 128), lambda i: (i, 0)),
            pl.BlockSpec((ROWS128, 128), lambda i: (0, 0)),
            pl.BlockSpec((ROWS128, 128), lambda i: (0, 0)),
        ],
        out_shape=[
            jax.ShapeDtypeStruct((ROWS128, 128), jnp.float32),
            jax.ShapeDtypeStruct((ROWS128, 128), jnp.int32),
            jax.ShapeDtypeStruct((ROWS128, 128), jnp.int32),
        ],
        scratch_shapes=[pltpu.VMEM((ROWS128, 128), jnp.float32)],
    )(data, W.reshape(1, D_FEAT), b.reshape(1, 1))


def _sc_pool(data, scores, g, cnt, structure):
    mesh = plsc.VectorSubcoreMesh(core_axis_name="c", subcore_axis_name="s")

    @functools.partial(
        pl.kernel,
        mesh=mesh,
        compiler_params=pltpu.CompilerParams(needs_layout_passes=False),
        out_type=[
            jax.ShapeDtypeStruct((K_KEEP, D_FEAT), jnp.float32),
            jax.ShapeDtypeStruct((2, N_EDGES), jnp.int32),
        ],
        scratch_types=[
            pltpu.VMEM((OUT_ROWS, D_FEAT), jnp.float32),
            pltpu.VMEM((NPAD,), jnp.int32),
            pltpu.VMEM((NPAD,), jnp.float32),
            pltpu.VMEM((NPAD,), jnp.int32),
            pltpu.VMEM((OUT_ROWS + 16,), jnp.int32),
            pltpu.VMEM((OUT_ROWS,), jnp.int32),
            pltpu.VMEM((2, EDGE_BIG), jnp.int32),
            pltpu.VMEM((2, EDGE_BIG), jnp.int32),
            pltpu.SemaphoreType.DMA,
            pltpu.SemaphoreType.DMA,
            pltpu.SemaphoreType.DMA,
            pltpu.SemaphoreType.DMA,
            pltpu.SemaphoreType.DMA,
            pltpu.SemaphoreType.DMA,
        ],
    )
    def k(data_hbm, scores_hbm, g_hbm, cnt_hbm, st_hbm,
          pooled_hbm, ps_hbm,
          dbuf, gt, scb, cb, kbuf, kidx, ebuf, ebo,
          sem_in, sem_e, sem_gt, sem_sc, sem_cnt, sem_eo):
        wid = lax.axis_index("s") * 2 + lax.axis_index("c")
        ob0 = ((wid * OUT_GROUPS8) // NTILES) * 8
        e0 = jnp.where(wid < 2, wid * EDGE_BIG,
                       2 * EDGE_BIG + (wid - 2) * EDGE_SMALL)

        @pl.when(wid < 2)
        def _():
            pltpu.make_async_copy(
                st_hbm.at[:, pl.ds(e0, EDGE_BIG)], ebuf, sem_e).start()

        @pl.when(wid >= 2)
        def _():
            pltpu.make_async_copy(
                st_hbm.at[:, pl.ds(e0, EDGE_SMALL)],
                ebuf.at[:, pl.ds(0, EDGE_SMALL)], sem_e).start()
        cp_gt = pltpu.make_async_copy(g_hbm, gt, sem_gt)
        cp_gt.start()
        cp_sc = pltpu.make_async_copy(scores_hbm, scb, sem_sc)
        cp_sc.start()
        cp_cnt = pltpu.make_async_copy(cnt_hbm, cb, sem_cnt)
        cp_cnt.start()

        cp_cnt.wait()

        def first_geq(target):
            lo = jnp.int32(0)
            for s in (8192, 4096, 2048, 1024, 512, 256, 128, 64, 32, 16,
                      8, 4, 2, 1):
                cand = lo + s
                idx = jnp.minimum(cand - 1, NPAD - 1)
                v = plsc.load_gather(cb, [lax.broadcast(idx, (16,))])[0]
                ok = (cand <= NPAD) & (v < target)
                lo = jnp.where(ok, cand, lo)
            return lo

        hi = ob0 + OUT_ROWS
        blo = first_geq(ob0 + 1) // 16
        bhi = first_geq(hi) // 16 + 1

        iota16 = lax.iota(jnp.int32, 16)
        cp_gt.wait()

        def scan_body(i, c):
            gv = gt[pl.ds(i * 16, 16)]
            m = (gv >= ob0) & (gv < hi)
            ids = lax.broadcast(i * 16, (16,)) + iota16
            plsc.store_compressed(kbuf.at[pl.ds(c, 16)], ids, mask=m)
            return c + plsc.all_reduce_population_count(m)[0]

        lax.fori_loop(blo, bhi, scan_body, 0)
        for j in range(OUT_ROWS // 16):
            kidx[pl.ds(j * 16, 16)] = kbuf[pl.ds(j * 16, 16)]

        pltpu.make_async_copy(data_hbm.at[kidx], dbuf, sem_in).start()

        @pl.when(wid < 2)
        def _():
            pltpu.make_async_copy(
                st_hbm.at[:, pl.ds(e0, EDGE_BIG)], ebuf, sem_e).wait()

        @pl.when(wid >= 2)
        def _():
            pltpu.make_async_copy(
                st_hbm.at[:, pl.ds(e0, EDGE_SMALL)],
                ebuf.at[:, pl.ds(0, EDGE_SMALL)], sem_e).wait()

        nvec = jnp.where(wid < 2, EDGE_BIG // 16, EDGE_SMALL // 16)

        def ebody(i, carry):
            sv = ebuf[0, pl.ds(i * 16, 16)]
            dv = ebuf[1, pl.ds(i * 16, 16)]
            a = plsc.load_gather(gt, [sv])
            bb = plsc.load_gather(gt, [dv])
            m = (a >= 0) & (bb >= 0)
            ebo[0, pl.ds(i * 16, 16)] = jnp.where(m, a, -1)
            ebo[1, pl.ds(i * 16, 16)] = jnp.where(m, bb, -1)
            return carry

        lax.fori_loop(0, nvec, ebody, 0)

        @pl.when(wid < 2)
        def _():
            pltpu.make_async_copy(
                ebo, ps_hbm.at[:, pl.ds(e0, EDGE_BIG)], sem_eo).start()

        @pl.when(wid >= 2)
        def _():
            pltpu.make_async_copy(
                ebo.at[:, pl.ds(0, EDGE_SMALL)],
                ps_hbm.at[:, pl.ds(e0, EDGE_SMALL)], sem_eo).start()

        pltpu.make_async_copy(data_hbm.at[kidx], dbuf, sem_in).wait()
        cp_sc.wait()

        def grp_body(j, carry):
            kv = kidx[pl.ds(j * 16, 16)]
            svec = plsc.load_gather(scb, [kv])
            for r in range(16):
                vv = lax.broadcast(svec[r], (16,))
                row = j * 16 + r
                for c in range(D_FEAT // 16):
                    dbuf[row, pl.ds(c * 16, 16)] = (
                        dbuf[row, pl.ds(c * 16, 16)] * vv)
            return carry

        lax.fori_loop(0, OUT_ROWS // 16, grp_body, 0)
        pltpu.sync_copy(dbuf, pooled_hbm.at[pl.ds(ob0, OUT_ROWS), :])

        @pl.when(wid < 2)
        def _():
            pltpu.make_async_copy(
                ebo, ps_hbm.at[:, pl.ds(e0, EDGE_BIG)], sem_eo).wait()

        @pl.when(wid >= 2)
        def _():
            pltpu.make_async_copy(
                ebo.at[:, pl.ds(0, EDGE_SMALL)],
                ps_hbm.at[:, pl.ds(e0, EDGE_SMALL)], sem_eo).wait()

    return k(data, scores, g, cnt, structure)


def kernel(data, structure, W, b):
    scores2d, g2d, cnt2d = _scores_select_tc(data, W, b)
    pooled, ps = _sc_pool(
        data, scores2d.reshape(NPAD), g2d.reshape(NPAD),
        cnt2d.reshape(NPAD), structure)
    return pooled, ps


# speedup vs baseline: 88.0032x; 1.0326x over previous
import functools

import jax
import jax.numpy as jnp
from jax import lax
from jax.experimental import pallas as pl
from jax.experimental.pallas import tpu as pltpu
from jax.experimental.pallas import tpu_sc as plsc

N_NODES = 10000
D_FEAT = 256
N_EDGES = 160000
NPAD = 10240
ROWS128 = NPAD // 128
K_KEEP = N_NODES // 2
NTILES = 32
OUT_ROWS = 160
OUT_GROUPS8 = K_KEEP // 8
EDGE_BIG = 5120
EDGE_SMALL = 4992


def _mv_sel_body(d_ref, w_ref, b_ref, s_ref, g_ref, c_ref, sb_ref):
    i = pl.program_id(0)
    wt = w_ref[...]
    for rr in range(40):
        blk = d_ref[pl.ds(rr * 128, 128), :]
        row = (
            lax.dot_general(wt, blk, (((1,), (1,)), ((), ())),
                            preferred_element_type=jnp.float32)
            + b_ref[0, 0]
        )
        s_ref[pl.ds(rr, 1), :] = row
        sb_ref[pl.ds(i * 40 + rr, 1), :] = row

    @pl.when(i == 1)
    def _():
        _sel_body(sb_ref, g_ref, c_ref)


def _sel_body(s_ref, g_ref, c_ref):
    s = s_ref[...]
    u = lax.bitcast_convert_type(s, jnp.uint32)
    key = jnp.where(u >= jnp.uint32(0x80000000), ~u, u | jnp.uint32(0x80000000))
    row = lax.broadcasted_iota(jnp.int32, (ROWS128, 128), 0)
    lane = lax.broadcasted_iota(jnp.int32, (ROWS128, 128), 1)
    flat = row * 128 + lane
    key = jnp.where(flat < N_NODES, key, jnp.uint32(0))

    def bs(i, ans):
        sh = jnp.uint32(31) - i.astype(jnp.uint32)
        cand = ans | (jnp.uint32(1) << sh)
        cnt = jnp.sum((key >= cand).astype(jnp.int32))
        return lax.select(cnt >= K_KEEP, cand, ans)

    thr = lax.fori_loop(0, 32, bs, jnp.uint32(0))
    cnt_gt = jnp.sum((key > thr).astype(jnp.int32))
    need = (K_KEEP - cnt_gt).astype(jnp.float32)
    eq = key == thr
    upper = (
        lax.broadcasted_iota(jnp.int32, (128, 128), 0)
        <= lax.broadcasted_iota(jnp.int32, (128, 128), 1)
    ).astype(jnp.float32)
    lstrict = (
        lax.broadcasted_iota(jnp.int32, (ROWS128, ROWS128), 0)
        > lax.broadcasted_iota(jnp.int32, (ROWS128, ROWS128), 1)
    ).astype(jnp.float32)
    ef = eq.astype(jnp.float32)
    incl_e = jnp.dot(ef, upper, preferred_element_type=jnp.float32)
    off_e = jnp.dot(lstrict, incl_e[:, 127:128], preferred_element_type=jnp.float32)
    excl_rank = incl_e - ef + off_e
    kept = (key > thr) | (eq & (excl_rank < need))
    kf = kept.astype(jnp.float32)
    incl_k = jnp.dot(kf, upper, preferred_element_type=jnp.float32)
    off_k = jnp.dot(lstrict, incl_k[:, 127:128], preferred_element_type=jnp.float32)
    cnt = (incl_k + off_k).astype(jnp.int32)
    g_ref[...] = jnp.where(kept, cnt - 1, -1)
    c_ref[...] = cnt


def _scores_select_tc(data, W, b):
    return pl.pallas_call(
        _mv_sel_body,
        grid=(2,),
        in_specs=[
            pl.BlockSpec((5120, D_FEAT), lambda i: (i, 0)),
            pl.BlockSpec((1, D_FEAT), lambda i: (0, 0)),
            pl.BlockSpec((1, 1), lambda i: (0, 0)),
        ],
        out_specs=[
            pl.BlockSpec((40, 128), lambda i: (i, 0)),
            pl.BlockSpec((ROWS128, 128), lambda i: (0, 0)),
            pl.BlockSpec((ROWS128, 128), lambda i: (0, 0)),
        ],
        out_shape=[
            jax.ShapeDtypeStruct((ROWS128, 128), jnp.float32),
            jax.ShapeDtypeStruct((ROWS128, 128), jnp.int32),
            jax.ShapeDtypeStruct((ROWS128, 128), jnp.int32),
        ],
        scratch_shapes=[pltpu.VMEM((ROWS128, 128), jnp.float32)],
    )(data, W.reshape(1, D_FEAT), b.reshape(1, 1))


def _sc_pool(data, scores, g, cnt, structure):
    mesh = plsc.VectorSubcoreMesh(core_axis_name="c", subcore_axis_name="s")

    @functools.partial(
        pl.kernel,
        mesh=mesh,
        compiler_params=pltpu.CompilerParams(needs_layout_passes=False),
        out_type=[
            jax.ShapeDtypeStruct((K_KEEP, D_FEAT), jnp.float32),
            jax.ShapeDtypeStruct((2, N_EDGES), jnp.int32),
        ],
        scratch_types=[
            pltpu.VMEM((OUT_ROWS, D_FEAT), jnp.float32),
            pltpu.VMEM((NPAD,), jnp.int32),
            pltpu.VMEM((NPAD,), jnp.float32),
            pltpu.VMEM((NPAD,), jnp.int32),
            pltpu.VMEM((OUT_ROWS + 16,), jnp.int32),
            pltpu.VMEM((OUT_ROWS,), jnp.int32),
            pltpu.VMEM((2, EDGE_BIG), jnp.int32),
            pltpu.VMEM((2, EDGE_BIG), jnp.int32),
            pltpu.SemaphoreType.DMA,
            pltpu.SemaphoreType.DMA,
            pltpu.SemaphoreType.DMA,
            pltpu.SemaphoreType.DMA,
            pltpu.SemaphoreType.DMA,
            pltpu.SemaphoreType.DMA,
        ],
    )
    def k(data_hbm, scores_hbm, g_hbm, cnt_hbm, st_hbm,
          pooled_hbm, ps_hbm,
          dbuf, gt, scb, cb, kbuf, kidx, ebuf, ebo,
          sem_in, sem_e, sem_gt, sem_sc, sem_cnt, sem_eo):
        wid = lax.axis_index("s") * 2 + lax.axis_index("c")
        ob0 = ((wid * OUT_GROUPS8) // NTILES) * 8
        e0 = jnp.where(wid < 2, wid * EDGE_BIG,
                       2 * EDGE_BIG + (wid - 2) * EDGE_SMALL)

        @pl.when(wid < 2)
        def _():
            pltpu.make_async_copy(
                st_hbm.at[:, pl.ds(e0, EDGE_BIG)], ebuf, sem_e).start()

        @pl.when(wid >= 2)
        def _():
            pltpu.make_async_copy(
                st_hbm.at[:, pl.ds(e0, EDGE_SMALL)],
                ebuf.at[:, pl.ds(0, EDGE_SMALL)], sem_e).start()
        cp_gt = pltpu.make_async_copy(g_hbm, gt, sem_gt)
        cp_gt.start()
        cp_sc = pltpu.make_async_copy(scores_hbm, scb, sem_sc)
        cp_sc.start()
        cp_cnt = pltpu.make_async_copy(cnt_hbm, cb, sem_cnt)
        cp_cnt.start()

        cp_cnt.wait()

        def first_geq(target):
            lo = jnp.int32(0)
            for s in (8192, 4096, 2048, 1024, 512, 256, 128, 64, 32, 16,
                      8, 4, 2, 1):
                cand = lo + s
                idx = jnp.minimum(cand - 1, NPAD - 1)
                v = plsc.load_gather(cb, [lax.broadcast(idx, (16,))])[0]
                ok = (cand <= NPAD) & (v < target)
                lo = jnp.where(ok, cand, lo)
            return lo

        hi = ob0 + OUT_ROWS
        blo = first_geq(ob0 + 1) // 16
        bhi = first_geq(hi) // 16 + 1

        iota16 = lax.iota(jnp.int32, 16)
        cp_gt.wait()

        def scan_body(i, c):
            gv = gt[pl.ds(i * 16, 16)]
            m = (gv >= ob0) & (gv < hi)
            ids = lax.broadcast(i * 16, (16,)) + iota16
            plsc.store_compressed(kbuf.at[pl.ds(c, 16)], ids, mask=m)
            return c + plsc.all_reduce_population_count(m)[0]

        lax.fori_loop(blo, bhi, scan_body, 0)
        for j in range(OUT_ROWS // 16):
            kidx[pl.ds(j * 16, 16)] = kbuf[pl.ds(j * 16, 16)]

        pltpu.make_async_copy(data_hbm.at[kidx], dbuf, sem_in).start()

        @pl.when(wid < 2)
        def _():
            pltpu.make_async_copy(
                st_hbm.at[:, pl.ds(e0, EDGE_BIG)], ebuf, sem_e).wait()

        @pl.when(wid >= 2)
        def _():
            pltpu.make_async_copy(
                st_hbm.at[:, pl.ds(e0, EDGE_SMALL)],
                ebuf.at[:, pl.ds(0, EDGE_SMALL)], sem_e).wait()

        nvec = jnp.where(wid < 2, EDGE_BIG // 16, EDGE_SMALL // 16)

        def ebody(i, carry):
            sv = ebuf[0, pl.ds(i * 16, 16)]
            dv = ebuf[1, pl.ds(i * 16, 16)]
            a = plsc.load_gather(gt, [sv])
            bb = plsc.load_gather(gt, [dv])
            m = (a >= 0) & (bb >= 0)
            ebo[0, pl.ds(i * 16, 16)] = jnp.where(m, a, -1)
            ebo[1, pl.ds(i * 16, 16)] = jnp.where(m, bb, -1)
            return carry

        lax.fori_loop(0, nvec, ebody, 0)

        @pl.when(wid < 2)
        def _():
            pltpu.make_async_copy(
                ebo, ps_hbm.at[:, pl.ds(e0, EDGE_BIG)], sem_eo).start()

        @pl.when(wid >= 2)
        def _():
            pltpu.make_async_copy(
                ebo.at[:, pl.ds(0, EDGE_SMALL)],
                ps_hbm.at[:, pl.ds(e0, EDGE_SMALL)], sem_eo).start()

        pltpu.make_async_copy(data_hbm.at[kidx], dbuf, sem_in).wait()
        cp_sc.wait()

        def grp_body(j, carry):
            kv = kidx[pl.ds(j * 16, 16)]
            svec = plsc.load_gather(scb, [kv])
            for r in range(16):
                vv = lax.broadcast(svec[r], (16,))
                row = j * 16 + r
                for c in range(D_FEAT // 16):
                    dbuf[row, pl.ds(c * 16, 16)] = (
                        dbuf[row, pl.ds(c * 16, 16)] * vv)
            return carry

        lax.fori_loop(0, OUT_ROWS // 16, grp_body, 0)
        pltpu.sync_copy(dbuf, pooled_hbm.at[pl.ds(ob0, OUT_ROWS), :])

        @pl.when(wid < 2)
        def _():
            pltpu.make_async_copy(
                ebo, ps_hbm.at[:, pl.ds(e0, EDGE_BIG)], sem_eo).wait()

        @pl.when(wid >= 2)
        def _():
            pltpu.make_async_copy(
                ebo.at[:, pl.ds(0, EDGE_SMALL)],
                ps_hbm.at[:, pl.ds(e0, EDGE_SMALL)], sem_eo).wait()

    return k(data, scores, g, cnt, structure)


def kernel(data, structure, W, b):
    scores2d, g2d, cnt2d = _scores_select_tc(data, W, b)
    pooled, ps = _sc_pool(
        data, scores2d.reshape(NPAD), g2d.reshape(NPAD),
        cnt2d.reshape(NPAD), structure)
    return pooled, ps


# speedup vs baseline: 92.0513x; 1.0460x over previous
import functools

import jax
import jax.numpy as jnp
from jax import lax
from jax.experimental import pallas as pl
from jax.experimental.pallas import tpu as pltpu
from jax.experimental.pallas import tpu_sc as plsc

N_NODES = 10000
D_FEAT = 256
N_EDGES = 160000
NPAD = 10240
ROWS128 = NPAD // 128
K_KEEP = N_NODES // 2
NTILES = 32
OUT_ROWS = 160
OUT_GROUPS8 = K_KEEP // 8
EDGE_BIG = 5120
EDGE_SMALL = 4992


def _mv_sel_body(d_ref, w_ref, b_ref, s_ref, g_ref, c_ref, sb_ref):
    i = pl.program_id(0)
    wt = w_ref[...]
    for rr in range(40):
        blk = d_ref[pl.ds(rr * 128, 128), :]
        row = (
            lax.dot_general(wt, blk, (((1,), (1,)), ((), ())),
                            preferred_element_type=jnp.float32)
            + b_ref[0, 0]
        )
        s_ref[pl.ds(rr, 1), :] = row
        sb_ref[pl.ds(i * 40 + rr, 1), :] = row

    @pl.when(i == 1)
    def _():
        _sel_body(sb_ref, g_ref, c_ref)


def _sel_body(s_ref, g_ref, c_ref):
    s = s_ref[...]
    u = lax.bitcast_convert_type(s, jnp.uint32)
    key = jnp.where(u >= jnp.uint32(0x80000000), ~u, u | jnp.uint32(0x80000000))
    row = lax.broadcasted_iota(jnp.int32, (ROWS128, 128), 0)
    lane = lax.broadcasted_iota(jnp.int32, (ROWS128, 128), 1)
    flat = row * 128 + lane
    key = jnp.where(flat < N_NODES, key, jnp.uint32(0))

    def bs(i, ans):
        sh = jnp.uint32(31) - i.astype(jnp.uint32)
        cand = ans | (jnp.uint32(1) << sh)
        cnt = jnp.sum((key >= cand).astype(jnp.int32))
        return lax.select(cnt >= K_KEEP, cand, ans)

    thr = lax.fori_loop(0, 32, bs, jnp.uint32(0))
    cnt_gt = jnp.sum((key > thr).astype(jnp.int32))
    need = (K_KEEP - cnt_gt).astype(jnp.float32)
    eq = key == thr
    upper = (
        lax.broadcasted_iota(jnp.int32, (128, 128), 0)
        <= lax.broadcasted_iota(jnp.int32, (128, 128), 1)
    ).astype(jnp.float32)
    lstrict = (
        lax.broadcasted_iota(jnp.int32, (ROWS128, ROWS128), 0)
        > lax.broadcasted_iota(jnp.int32, (ROWS128, ROWS128), 1)
    ).astype(jnp.float32)
    ef = eq.astype(jnp.float32)
    incl_e = jnp.dot(ef, upper, preferred_element_type=jnp.float32)
    off_e = jnp.dot(lstrict, incl_e[:, 127:128], preferred_element_type=jnp.float32)
    excl_rank = incl_e - ef + off_e
    kept = (key > thr) | (eq & (excl_rank < need))
    kf = kept.astype(jnp.float32)
    incl_k = jnp.dot(kf, upper, preferred_element_type=jnp.float32)
    off_k = jnp.dot(lstrict, incl_k[:, 127:128], preferred_element_type=jnp.float32)
    cnt = (incl_k + off_k).astype(jnp.int32)
    g_ref[...] = jnp.where(kept, cnt - 1, -1)
    c_ref[...] = cnt


def _scores_select_tc(data, W, b):
    return pl.pallas_call(
        _mv_sel_body,
        grid=(2,),
        in_specs=[
            pl.BlockSpec((5120, D_FEAT), lambda i: (i, 0)),
            pl.BlockSpec((1, D_FEAT), lambda i: (0, 0)),
            pl.BlockSpec((1, 1), lambda i: (0, 0)),
        ],
        out_specs=[
            pl.BlockSpec((40, 128), lambda i: (i, 0)),
            pl.BlockSpec((ROWS128, 128), lambda i: (0, 0)),
            pl.BlockSpec((ROWS128, 128), lambda i: (0, 0)),
        ],
        out_shape=[
            jax.ShapeDtypeStruct((ROWS128, 128), jnp.float32),
            jax.ShapeDtypeStruct((ROWS128, 128), jnp.int32),
            jax.ShapeDtypeStruct((ROWS128, 128), jnp.int32),
        ],
        scratch_shapes=[pltpu.VMEM((ROWS128, 128), jnp.float32)],
    )(data, W.reshape(1, D_FEAT), b.reshape(1, 1))


def _sc_pool(data, scores, g, cnt, structure):
    mesh = plsc.VectorSubcoreMesh(core_axis_name="c", subcore_axis_name="s")

    @functools.partial(
        pl.kernel,
        mesh=mesh,
        compiler_params=pltpu.CompilerParams(needs_layout_passes=False),
        out_type=[
            jax.ShapeDtypeStruct((K_KEEP, D_FEAT), jnp.float32),
            jax.ShapeDtypeStruct((2, N_EDGES), jnp.int32),
        ],
        scratch_types=[
            pltpu.VMEM((OUT_ROWS, D_FEAT), jnp.float32),
            pltpu.VMEM((NPAD,), jnp.int32),
            pltpu.VMEM((NPAD,), jnp.float32),
            pltpu.VMEM((NPAD,), jnp.int32),
            pltpu.VMEM((OUT_ROWS + 16,), jnp.int32),
            pltpu.VMEM((OUT_ROWS,), jnp.int32),
            pltpu.VMEM((2, EDGE_BIG), jnp.int32),
            pltpu.VMEM((2, EDGE_BIG), jnp.int32),
            pltpu.SemaphoreType.DMA,
            pltpu.SemaphoreType.DMA,
            pltpu.SemaphoreType.DMA,
            pltpu.SemaphoreType.DMA,
            pltpu.SemaphoreType.DMA,
            pltpu.SemaphoreType.DMA,
        ],
    )
    def k(data_hbm, scores_hbm, g_hbm, cnt_hbm, st_hbm,
          pooled_hbm, ps_hbm,
          dbuf, gt, scb, cb, kbuf, kidx, ebuf, ebo,
          sem_in, sem_e, sem_gt, sem_sc, sem_cnt, sem_eo):
        wid = lax.axis_index("s") * 2 + lax.axis_index("c")
        ob0 = ((wid * OUT_GROUPS8) // NTILES) * 8
        e0 = jnp.where(wid < 2, wid * EDGE_BIG,
                       2 * EDGE_BIG + (wid - 2) * EDGE_SMALL)

        @pl.when(wid < 2)
        def _():
            pltpu.make_async_copy(
                st_hbm.at[:, pl.ds(e0, EDGE_BIG)], ebuf, sem_e).start()

        @pl.when(wid >= 2)
        def _():
            pltpu.make_async_copy(
                st_hbm.at[:, pl.ds(e0, EDGE_SMALL)],
                ebuf.at[:, pl.ds(0, EDGE_SMALL)], sem_e).start()
        cp_gt = pltpu.make_async_copy(g_hbm, gt, sem_gt)
        cp_gt.start()
        cp_sc = pltpu.make_async_copy(scores_hbm, scb, sem_sc)
        cp_sc.start()
        cp_cnt = pltpu.make_async_copy(cnt_hbm, cb, sem_cnt)
        cp_cnt.start()

        cp_cnt.wait()

        def first_geq(target):
            lo = jnp.int32(0)
            for s in (8192, 4096, 2048, 1024, 512, 256, 128, 64, 32, 16,
                      8, 4, 2, 1):
                cand = lo + s
                idx = jnp.minimum(cand - 1, NPAD - 1)
                v = plsc.load_gather(cb, [lax.broadcast(idx, (16,))])[0]
                ok = (cand <= NPAD) & (v < target)
                lo = jnp.where(ok, cand, lo)
            return lo

        hi = ob0 + OUT_ROWS
        blo = first_geq(ob0 + 1) // 16
        bhi = first_geq(hi) // 16 + 1

        iota16 = lax.iota(jnp.int32, 16)
        cp_gt.wait()

        def scan_body(i, c):
            gv = gt[pl.ds(i * 16, 16)]
            m = (gv >= ob0) & (gv < hi)
            ids = lax.broadcast(i * 16, (16,)) + iota16
            plsc.store_compressed(kbuf.at[pl.ds(c, 16)], ids, mask=m)
            return c + plsc.all_reduce_population_count(m)[0]

        lax.fori_loop(blo, bhi, scan_body, 0)
        for j in range(OUT_ROWS // 16):
            kidx[pl.ds(j * 16, 16)] = kbuf[pl.ds(j * 16, 16)]

        pltpu.make_async_copy(data_hbm.at[kidx], dbuf, sem_in).start()

        @pl.when(wid < 2)
        def _():
            pltpu.make_async_copy(
                st_hbm.at[:, pl.ds(e0, EDGE_BIG)], ebuf, sem_e).wait()

        @pl.when(wid >= 2)
        def _():
            pltpu.make_async_copy(
                st_hbm.at[:, pl.ds(e0, EDGE_SMALL)],
                ebuf.at[:, pl.ds(0, EDGE_SMALL)], sem_e).wait()

        nvec = jnp.where(wid < 2, EDGE_BIG // 16, EDGE_SMALL // 16)

        @functools.partial(plsc.parallel_loop, 0, nvec, unroll=4)
        def _(i):
            sv = ebuf[0, pl.ds(i * 16, 16)]
            dv = ebuf[1, pl.ds(i * 16, 16)]
            a = plsc.load_gather(gt, [sv])
            bb = plsc.load_gather(gt, [dv])
            m = (a >= 0) & (bb >= 0)
            ebo[0, pl.ds(i * 16, 16)] = jnp.where(m, a, -1)
            ebo[1, pl.ds(i * 16, 16)] = jnp.where(m, bb, -1)

        @pl.when(wid < 2)
        def _():
            pltpu.make_async_copy(
                ebo, ps_hbm.at[:, pl.ds(e0, EDGE_BIG)], sem_eo).start()

        @pl.when(wid >= 2)
        def _():
            pltpu.make_async_copy(
                ebo.at[:, pl.ds(0, EDGE_SMALL)],
                ps_hbm.at[:, pl.ds(e0, EDGE_SMALL)], sem_eo).start()

        pltpu.make_async_copy(data_hbm.at[kidx], dbuf, sem_in).wait()
        cp_sc.wait()

        @functools.partial(plsc.parallel_loop, 0, OUT_ROWS // 16)
        def _(j):
            kv = kidx[pl.ds(j * 16, 16)]
            svec = plsc.load_gather(scb, [kv])
            for r in range(16):
                vv = lax.broadcast(svec[r], (16,))
                row = j * 16 + r
                for c in range(D_FEAT // 16):
                    dbuf[row, pl.ds(c * 16, 16)] = (
                        dbuf[row, pl.ds(c * 16, 16)] * vv)
        pltpu.sync_copy(dbuf, pooled_hbm.at[pl.ds(ob0, OUT_ROWS), :])

        @pl.when(wid < 2)
        def _():
            pltpu.make_async_copy(
                ebo, ps_hbm.at[:, pl.ds(e0, EDGE_BIG)], sem_eo).wait()

        @pl.when(wid >= 2)
        def _():
            pltpu.make_async_copy(
                ebo.at[:, pl.ds(0, EDGE_SMALL)],
                ps_hbm.at[:, pl.ds(e0, EDGE_SMALL)], sem_eo).wait()

    return k(data, scores, g, cnt, structure)


def kernel(data, structure, W, b):
    scores2d, g2d, cnt2d = _scores_select_tc(data, W, b)
    pooled, ps = _sc_pool(
        data, scores2d.reshape(NPAD), g2d.reshape(NPAD),
        cnt2d.reshape(NPAD), structure)
    return pooled, ps
